# baseline (speedup 1.0000x reference)
import functools
import jax
import jax.numpy as jnp
from jax import lax
from jax.experimental import pallas as pl
from jax.experimental.pallas import tpu as pltpu, tpu_sc as plsc

NS = 50000
NT = 10000
NU = 5000
NI = 5000
E1 = 500000
E2 = 320000
E3 = 320000
VOCAB = 200000
D = 128
L = 16
NC = 2
NSC = 16
NW = NC * NSC

NSp = 50176
NTp = 10240
NUp = 5120
NIp = 5120
E1p = 507904
E2p = 327680
E3p = 327680

_mesh = plsc.VectorSubcoreMesh(core_axis_name="c", subcore_axis_name="s")


def _wid():
    return lax.axis_index("s") * NC + lax.axis_index("c")


def _zero_rows(buf, nrows):
    def zr(i, _):
        for j in range(D // L):
            buf[i, pl.ds(j * L, L)] = jnp.zeros((L,), jnp.float32)
        return 0
    lax.fori_loop(0, nrows, zr, 0)


def _zero_shared(buf, sh, s, rows_per_sub):
    if rows_per_sub <= 64:
        pltpu.sync_copy(buf.at[pl.ds(0, rows_per_sub)],
                        sh.at[pl.ds(s * rows_per_sub, rows_per_sub)])
    else:
        def zc(i, _):
            pltpu.sync_copy(buf.at[pl.ds(0, 64)],
                            sh.at[pl.ds(s * rows_per_sub + i * 64, 64)])
            return 0
        lax.fori_loop(0, rows_per_sub // 64, zc, 0)


def _writeback(buf, sh, out, c, s, rows_per_sub):
    if rows_per_sub <= 64:
        pltpu.sync_copy(sh.at[pl.ds(s * rows_per_sub, rows_per_sub)],
                        buf.at[pl.ds(0, rows_per_sub)])
        pltpu.sync_copy(buf.at[pl.ds(0, rows_per_sub)],
                        out.at[c, pl.ds(s * rows_per_sub, rows_per_sub)])
    else:
        def wc(i, _):
            base = s * rows_per_sub + i * 64
            pltpu.sync_copy(sh.at[pl.ds(base, 64)], buf.at[pl.ds(0, 64)])
            pltpu.sync_copy(buf.at[pl.ds(0, 64)], out.at[c, pl.ds(base, 64)])
            return 0
        lax.fori_loop(0, rows_per_sub // 64, wc, 0)


@functools.partial(
    pl.kernel,
    out_type=(jax.ShapeDtypeStruct((NSp, D), jnp.float32),
              jax.ShapeDtypeStruct((NTp, D), jnp.float32),
              jax.ShapeDtypeStruct((NTp, D), jnp.float32)),
    mesh=_mesh,
    scratch_types=[pltpu.VMEM((1, 128), jnp.int32),
                   pltpu.VMEM((1, 128), jnp.int32),
                   pltpu.VMEM((128, D), jnp.float32),
                   pltpu.VMEM((128, D), jnp.float32),
                   pltpu.SemaphoreType.DMA,
                   pltpu.SemaphoreType.DMA],
)
def _k1(emb, w1, tuw, tiw, sids2d, stid2d, su2d, si2d,
        h_out, wu_out, wi_out, ia, ib, ra, rb, semA, semB):
    w = _wid()
    nch_h = NSp // 128

    def hchunk(t, _):
        ch = t * NW + w

        @pl.when(ch < nch_h)
        def _():
            pltpu.sync_copy(sids2d.at[pl.ds(ch, 1)], ia)
            pltpu.sync_copy(stid2d.at[pl.ds(ch, 1)], ib)
            pltpu.async_copy(emb.at[ia.at[0]], ra, semA)
            pltpu.async_copy(w1.at[ib.at[0]], rb, semB)
            pltpu.make_async_copy(emb.at[ia.at[0]], ra, semA).wait()
            pltpu.make_async_copy(w1.at[ib.at[0]], rb, semB).wait()

            def mrow(kk, _):
                for j in range(D // L):
                    ra[kk, pl.ds(j * L, L)] = (ra[kk, pl.ds(j * L, L)] *
                                               rb[kk, pl.ds(j * L, L)])
                return 0
            lax.fori_loop(0, 128, mrow, 0)
            pltpu.sync_copy(ra, h_out.at[pl.ds(ch * 128, 128)])
        return 0
    lax.fori_loop(0, nch_h // NW + 1, hchunk, 0)

    nch_t = NTp // 128

    def tchunk(t, _):
        ch = t * NW + w

        @pl.when(ch < nch_t)
        def _():
            pltpu.sync_copy(su2d.at[pl.ds(ch, 1)], ia)
            pltpu.sync_copy(si2d.at[pl.ds(ch, 1)], ib)
            pltpu.async_copy(tuw.at[ia.at[0]], ra, semA)
            pltpu.async_copy(tiw.at[ib.at[0]], rb, semB)
            pltpu.make_async_copy(tuw.at[ia.at[0]], ra, semA).wait()
            pltpu.sync_copy(ra, wu_out.at[pl.ds(ch * 128, 128)])
            pltpu.make_async_copy(tiw.at[ib.at[0]], rb, semB).wait()
            pltpu.sync_copy(rb, wi_out.at[pl.ds(ch * 128, 128)])
        return 0
    lax.fori_loop(0, nch_t // NW + 1, tchunk, 0)


def _make_passA(ndst, nchunks):
    cpw = nchunks // NW

    @functools.partial(
        pl.kernel,
        out_type=jax.ShapeDtypeStruct((NC, ndst, D), jnp.float32),
        mesh=_mesh,
        scratch_types=[pltpu.VMEM((1, 128), jnp.int32),
                       pltpu.VMEM((1, 128), jnp.int32),
                       pltpu.VMEM((1, 128), jnp.int32),
                       pltpu.VMEM((1, 128), jnp.int32),
                       pltpu.VMEM((128, D), jnp.float32),
                       pltpu.VMEM((128, D), jnp.float32),
                       pltpu.VMEM_SHARED((ndst, D), jnp.float32),
                       pltpu.SemaphoreType.DMA,
                       pltpu.SemaphoreType.DMA],
    )
    def kA(h, src2d, dst2d, accp,
           si0, di0, si1, di1, rows0, rows1, acc_sh, semA, semB):
        c, s = lax.axis_index("c"), lax.axis_index("s")
        w = s * NC + c
        _zero_rows(rows0, 64)
        _zero_shared(rows0, acc_sh, s, ndst // NSC)
        plsc.subcore_barrier()
        base = w * cpw
        pltpu.sync_copy(src2d.at[pl.ds(base, 1)], si0)
        pltpu.sync_copy(dst2d.at[pl.ds(base, 1)], di0)
        pltpu.async_copy(h.at[si0.at[0]], rows0, semA)

        def pair(t2, _):
            a = base + 2 * t2
            pltpu.sync_copy(src2d.at[pl.ds(a + 1, 1)], si1)
            pltpu.sync_copy(dst2d.at[pl.ds(a + 1, 1)], di1)
            pltpu.async_copy(h.at[si1.at[0]], rows1, semB)
            pltpu.make_async_copy(h.at[si0.at[0]], rows0, semA).wait()
            pltpu.sync_copy(rows0, acc_sh.at[di0.at[0]], add=True)

            @pl.when(t2 + 1 < cpw // 2)
            def _():
                pltpu.sync_copy(src2d.at[pl.ds(a + 2, 1)], si0)
                pltpu.sync_copy(dst2d.at[pl.ds(a + 2, 1)], di0)
                pltpu.async_copy(h.at[si0.at[0]], rows0, semA)
            pltpu.make_async_copy(h.at[si1.at[0]], rows1, semB).wait()
            pltpu.sync_copy(rows1, acc_sh.at[di1.at[0]], add=True)
            return 0
        lax.fori_loop(0, cpw // 2, pair, 0)
        plsc.subcore_barrier()
        _writeback(rows0, acc_sh, accp, c, s, ndst // NSC)

    return kA


def _make_combine(ndst):
    nch = ndst // 128

    @functools.partial(
        pl.kernel,
        out_type=jax.ShapeDtypeStruct((ndst, D), jnp.float32),
        mesh=_mesh,
        scratch_types=[pltpu.VMEM((128, D), jnp.float32),
                       pltpu.VMEM((128, D), jnp.float32)],
    )
    def kC(accp, out, ra, rb):
        w = _wid()

        def chunk(t, _):
            ch = t * NW + w

            @pl.when(ch < nch)
            def _():
                pltpu.sync_copy(accp.at[0, pl.ds(ch * 128, 128)], ra)
                pltpu.sync_copy(accp.at[1, pl.ds(ch * 128, 128)], rb)

                def arow(kk, _):
                    for j in range(D // L):
                        ra[kk, pl.ds(j * L, L)] = (ra[kk, pl.ds(j * L, L)] +
                                                   rb[kk, pl.ds(j * L, L)])
                    return 0
                lax.fori_loop(0, 128, arow, 0)
                pltpu.sync_copy(ra, out.at[pl.ds(ch * 128, 128)])
            return 0
        lax.fori_loop(0, nch // NW + 1, chunk, 0)

    return kC


def _make_passBC(ndst, nchunks):
    cpw = nchunks // NW
    nz = ndst // 128
    nzs = 8
    nzp = nzs * NSC
    assert nz <= nzp

    @functools.partial(
        pl.kernel,
        out_type=(jax.ShapeDtypeStruct((NC, ndst, D), jnp.float32),
                  jax.ShapeDtypeStruct((NC, nzp, D), jnp.float32)),
        mesh=_mesh,
        scratch_types=[pltpu.VMEM((1, 128), jnp.int32),
                       pltpu.VMEM((1, 128), jnp.int32),
                       pltpu.VMEM((1, 128), jnp.int32),
                       pltpu.VMEM((8, L), jnp.int32),
                       pltpu.VMEM((8, L), jnp.float32),
                       pltpu.VMEM((128, D), jnp.float32),
                       pltpu.VMEM((128, D), jnp.float32),
                       pltpu.VMEM_SHARED((ndst, D), jnp.float32),
                       pltpu.VMEM_SHARED((nzp, D), jnp.float32),
                       pltpu.SemaphoreType.DMA,
                       pltpu.SemaphoreType.DMA],
    )
    def kBC(h, sumh, src2d, dst2d, outp, zp,
            si, di, zi, lo78, ebuf, rh, rs, out_sh, z_sh, semA, semB):
        c, s = lax.axis_index("c"), lax.axis_index("s")
        w = s * NC + c
        _zero_rows(rh, 64)
        _zero_shared(rh, out_sh, s, ndst // NSC)
        pltpu.sync_copy(rh.at[pl.ds(0, nzs)], z_sh.at[pl.ds(s * nzs, nzs)])
        plsc.subcore_barrier()
        lanes = lax.broadcasted_iota(jnp.int32, (L,), 0)

        def step(t, _):
            row = w * cpw + t
            pltpu.sync_copy(src2d.at[pl.ds(row, 1)], si)
            pltpu.sync_copy(dst2d.at[pl.ds(row, 1)], di)
            pltpu.async_copy(h.at[si.at[0]], rh, semA)
            pltpu.async_copy(sumh.at[di.at[0]], rs, semB)
            pltpu.make_async_copy(h.at[si.at[0]], rh, semA).wait()
            pltpu.make_async_copy(sumh.at[di.at[0]], rs, semB).wait()
            for j in range(128 // L):
                dv = di[0, pl.ds(j * L, L)]
                zi[0, pl.ds(j * L, L)] = jnp.right_shift(dv, 7)
                lo78[j, pl.ds(0, L)] = dv & 127

            @plsc.parallel_loop(0, 128 // L, unroll=2)
            def group(g):
                onehots = []
                for k2 in range(L):
                    kk = g * L + k2
                    acc0 = rh[kk, pl.ds(0, L)] * rs[kk, pl.ds(0, L)]
                    acc1 = rh[kk, pl.ds(4 * L, L)] * rs[kk, pl.ds(4 * L, L)]
                    for j in range(1, 4):
                        acc0 = acc0 + (rh[kk, pl.ds(j * L, L)] *
                                       rs[kk, pl.ds(j * L, L)])
                        acc1 = acc1 + (rh[kk, pl.ds((j + 4) * L, L)] *
                                       rs[kk, pl.ds((j + 4) * L, L)])
                    acc = acc0 + acc1
                    e = [acc[l] for l in range(L)]
                    while len(e) > 1:
                        e = [e[i] + e[i + 1] for i in range(0, len(e), 2)]
                    onehots.append(jnp.where(lanes == k2, e[0], 0.0))
                while len(onehots) > 1:
                    onehots = [onehots[i] + onehots[i + 1]
                               for i in range(0, len(onehots), 2)]
                ev = jnp.exp(onehots[0])
                ebuf[g, pl.ds(0, L)] = ev
                for k2 in range(L):
                    kk = g * L + k2
                    e = ev[k2]
                    for j in range(D // L):
                        rh[kk, pl.ds(j * L, L)] = rh[kk, pl.ds(j * L, L)] * e
            pltpu.sync_copy(rh, out_sh.at[di.at[0]], add=True)

            @plsc.parallel_loop(0, 128 // L, unroll=2)
            def zgroup(g):
                ev = ebuf[g, pl.ds(0, L)]
                lv = lo78[g, pl.ds(0, L)]
                for k2 in range(L):
                    kk = g * L + k2
                    e = ev[k2]
                    low7 = lv[k2]
                    for j in range(D // L):
                        rh[kk, pl.ds(j * L, L)] = jnp.where(
                            lanes + (j * L) == low7, e, 0.0)
            pltpu.sync_copy(rh, z_sh.at[zi.at[0]], add=True)
            return 0
        lax.fori_loop(0, cpw, step, 0)
        plsc.subcore_barrier()
        _writeback(rh, out_sh, outp, c, s, ndst // NSC)
        pltpu.sync_copy(z_sh.at[pl.ds(s * nzs, nzs)], rh.at[pl.ds(0, nzs)])
        pltpu.sync_copy(rh.at[pl.ds(0, nzs)], zp.at[c, pl.ds(s * nzs, nzs)])

    return kBC


_k2 = _make_passA(NTp, E1p // 128)
_k3 = _make_combine(NTp)
_k4 = _make_passBC(NTp, E1p // 128)
_c23 = _make_combine(NUp)


@functools.partial(
    pl.kernel,
    out_type=(jax.ShapeDtypeStruct((NC, NUp, D), jnp.float32),
              jax.ShapeDtypeStruct((NC, NIp, D), jnp.float32)),
    mesh=_mesh,
    scratch_types=[pltpu.VMEM((1, 128), jnp.int32),
                   pltpu.VMEM((1, 128), jnp.int32),
                   pltpu.VMEM((1, 128), jnp.int32),
                   pltpu.VMEM((1, 128), jnp.int32),
                   pltpu.VMEM((128, D), jnp.float32),
                   pltpu.VMEM((128, D), jnp.float32),
                   pltpu.VMEM_SHARED((NUp, D), jnp.float32),
                   pltpu.VMEM_SHARED((NIp, D), jnp.float32),
                   pltpu.SemaphoreType.DMA,
                   pltpu.SemaphoreType.DMA],
)
def _k6(hu, hi, src2_2d, dst2_2d, src3_2d, dst3_2d, accu, acci,
        si0, di0, si1, di1, rows0, rows1, accu_sh, acci_sh, semA, semB):
    c, s = lax.axis_index("c"), lax.axis_index("s")
    w = s * NC + c
    cpw = (E2p // 128) // NW
    _zero_rows(rows0, 64)
    _zero_shared(rows0, accu_sh, s, NUp // NSC)
    _zero_shared(rows0, acci_sh, s, NIp // NSC)
    plsc.subcore_barrier()

    def scan_edges(tbl, src2d, dst2d, sh):
        base = w * cpw
        pltpu.sync_copy(src2d.at[pl.ds(base, 1)], si0)
        pltpu.sync_copy(dst2d.at[pl.ds(base, 1)], di0)
        pltpu.async_copy(tbl.at[si0.at[0]], rows0, semA)

        def pair(t2, _):
            a = base + 2 * t2
            pltpu.sync_copy(src2d.at[pl.ds(a + 1, 1)], si1)
            pltpu.sync_copy(dst2d.at[pl.ds(a + 1, 1)], di1)
            pltpu.async_copy(tbl.at[si1.at[0]], rows1, semB)
            pltpu.make_async_copy(tbl.at[si0.at[0]], rows0, semA).wait()
            pltpu.sync_copy(rows0, sh.at[di0.at[0]], add=True)

            @pl.when(t2 + 1 < cpw // 2)
            def _():
                pltpu.sync_copy(src2d.at[pl.ds(a + 2, 1)], si0)
                pltpu.sync_copy(dst2d.at[pl.ds(a + 2, 1)], di0)
                pltpu.async_copy(tbl.at[si0.at[0]], rows0, semA)
            pltpu.make_async_copy(tbl.at[si1.at[0]], rows1, semB).wait()
            pltpu.sync_copy(rows1, sh.at[di1.at[0]], add=True)
            return 0
        lax.fori_loop(0, cpw // 2, pair, 0)

    scan_edges(hu, src2_2d, dst2_2d, accu_sh)
    scan_edges(hi, src3_2d, dst3_2d, acci_sh)
    plsc.subcore_barrier()
    _writeback(rows0, accu_sh, accu, c, s, NUp // NSC)
    _writeback(rows0, acci_sh, acci, c, s, NIp // NSC)


_k8u = _make_passBC(NUp, E2p // 128)
_k8i = _make_passBC(NIp, E3p // 128)


def _gelu_exact(x):
    return 0.5 * x * (1.0 + lax.erf(x * 0.7071067811865476))


def _k5_body(op_ref, zp_ref, w_ref, b_ref, wu_ref, wi_ref, hu_ref, hi_ref):
    t = op_ref[0] + op_ref[1]
    z = zp_ref[0] + zp_ref[1]
    tn = t / (z + 1e-9)
    tf = jnp.dot(tn, w_ref[...].T, preferred_element_type=jnp.float32) + b_ref[...]
    hu_ref[...] = _gelu_exact(tf * wu_ref[...])
    hi_ref[...] = _gelu_exact(tf * wi_ref[...])


def _k5(outp, zp2d, w, b2d, wu_rows, wi_rows):
    nblk = NTp // 128
    return pl.pallas_call(
        _k5_body,
        grid=(nblk,),
        in_specs=[
            pl.BlockSpec((NC, 128, D), lambda i: (0, i, 0)),
            pl.BlockSpec((NC, 128, 1), lambda i: (0, i, 0)),
            pl.BlockSpec((D, D), lambda i: (0, 0)),
            pl.BlockSpec((1, D), lambda i: (0, 0)),
            pl.BlockSpec((128, D), lambda i: (i, 0)),
            pl.BlockSpec((128, D), lambda i: (i, 0)),
        ],
        out_specs=[pl.BlockSpec((128, D), lambda i: (i, 0)),
                   pl.BlockSpec((128, D), lambda i: (i, 0))],
        out_shape=[jax.ShapeDtypeStruct((NTp, D), jnp.float32),
                   jax.ShapeDtypeStruct((NTp, D), jnp.float32)],
    )(outp, zp2d, w, b2d, wu_rows, wi_rows)


def _k9_body(up_ref, zu_ref, ip_ref, zi_ref, wu_ref, bu_ref, wi_ref, bi_ref,
             uf_ref, if_ref):
    u = (up_ref[0] + up_ref[1]) / (zu_ref[0] + zu_ref[1] + 1e-9)
    uf_ref[...] = jnp.dot(u, wu_ref[...].T,
                          preferred_element_type=jnp.float32) + bu_ref[...]
    v = (ip_ref[0] + ip_ref[1]) / (zi_ref[0] + zi_ref[1] + 1e-9)
    if_ref[...] = jnp.dot(v, wi_ref[...].T,
                          preferred_element_type=jnp.float32) + bi_ref[...]


def _k9(up, zu2d, ip, zi2d, u_w, u_b2d, i_w, i_b2d):
    nblk = NUp // 128
    return pl.pallas_call(
        _k9_body,
        grid=(nblk,),
        in_specs=[
            pl.BlockSpec((NC, 128, D), lambda i: (0, i, 0)),
            pl.BlockSpec((NC, 128, 1), lambda i: (0, i, 0)),
            pl.BlockSpec((NC, 128, D), lambda i: (0, i, 0)),
            pl.BlockSpec((NC, 128, 1), lambda i: (0, i, 0)),
            pl.BlockSpec((D, D), lambda i: (0, 0)),
            pl.BlockSpec((1, D), lambda i: (0, 0)),
            pl.BlockSpec((D, D), lambda i: (0, 0)),
            pl.BlockSpec((1, D), lambda i: (0, 0)),
        ],
        out_specs=[pl.BlockSpec((128, D), lambda i: (i, 0)),
                   pl.BlockSpec((128, D), lambda i: (i, 0))],
        out_shape=[jax.ShapeDtypeStruct((NUp, D), jnp.float32),
                   jax.ShapeDtypeStruct((NIp, D), jnp.float32)],
    )(up, zu2d, ip, zi2d, u_w, u_b2d, i_w, i_b2d)


def _pad_idx(x, n, mod):
    extra = n - x.shape[0]
    fill = jnp.arange(extra, dtype=jnp.int32) % mod
    return jnp.concatenate([x.astype(jnp.int32), fill])


def _pad_dst(x, n, real, padspace):
    extra = n - x.shape[0]
    fill = real + (jnp.arange(extra, dtype=jnp.int32) % padspace)
    return jnp.concatenate([x.astype(jnp.int32), fill])


def kernel(emb_table, sentence_w1, sent_lin_w, sent_lin_b, user_lin_w,
           user_lin_b, item_lin_w, item_lin_b, topic_user_w, topic_item_w,
           sentence_ids, stid_sent, src1, dst1, stid_user, src2, dst2,
           stid_item, src3, dst3):
    sids2d = _pad_idx(sentence_ids, NSp, VOCAB).reshape(-1, 128)
    stid2d = _pad_idx(stid_sent, NSp, 1024).reshape(-1, 128)
    su2d = _pad_idx(stid_user, NTp, 1024).reshape(-1, 128)
    si2d = _pad_idx(stid_item, NTp, 1024).reshape(-1, 128)
    src1_2d = _pad_idx(src1, E1p, NS).reshape(-1, 128)
    dst1_2d = _pad_dst(dst1, E1p, NT, NTp - NT).reshape(-1, 128)
    src2_2d = _pad_idx(src2, E2p, NT).reshape(-1, 128)
    dst2_2d = _pad_dst(dst2, E2p, NU, NUp - NU).reshape(-1, 128)
    src3_2d = _pad_idx(src3, E3p, NT).reshape(-1, 128)
    dst3_2d = _pad_dst(dst3, E3p, NI, NIp - NI).reshape(-1, 128)

    h, wu_rows, wi_rows = _k1(emb_table, sentence_w1, topic_user_w,
                              topic_item_w, sids2d, stid2d, su2d, si2d)

    sumh_p = _k2(h, src1_2d, dst1_2d)
    sumh = _k3(sumh_p)
    outp, zp = _k4(h, sumh, src1_2d, dst1_2d)
    zp2d = zp.reshape(NC, -1)[:, :NTp].reshape(NC, NTp, 1)
    hu, hi = _k5(outp, zp2d, sent_lin_w, sent_lin_b.reshape(1, D),
                 wu_rows, wi_rows)

    accu_p, acci_p = _k6(hu, hi, src2_2d, dst2_2d, src3_2d, dst3_2d)
    sumh2 = _c23(accu_p)
    sumh3 = _c23(acci_p)
    up, zu = _k8u(hu, sumh2, src2_2d, dst2_2d)
    ip, zi = _k8i(hi, sumh3, src3_2d, dst3_2d)
    zu2d = zu.reshape(NC, -1)[:, :NUp].reshape(NC, NUp, 1)
    zi2d = zi.reshape(NC, -1)[:, :NIp].reshape(NC, NIp, 1)
    user_feat, item_feat = _k9(up, zu2d, ip, zi2d,
                               user_lin_w, user_lin_b.reshape(1, D),
                               item_lin_w, item_lin_b.reshape(1, D))
    return (user_feat[:NU], item_feat[:NI])


# speedup vs baseline: 1.5693x; 1.5693x over previous
import functools
import jax
import jax.numpy as jnp
from jax import lax
from jax.experimental import pallas as pl
from jax.experimental.pallas import tpu as pltpu, tpu_sc as plsc

NS = 50000
NT = 10000
NU = 5000
NI = 5000
E1 = 500000
E2 = 320000
E3 = 320000
VOCAB = 200000
D = 128
L = 16
NC = 2
NSC = 16
NW = NC * NSC

NSp = 50176
NTp = 10240
NUp = 5120
NIp = 5120
E1p = 507904
E2p = 327680
E3p = 327680

_mesh = plsc.VectorSubcoreMesh(core_axis_name="c", subcore_axis_name="s")


def _wid():
    return lax.axis_index("s") * NC + lax.axis_index("c")


def _zero_rows(buf, nrows):
    def zr(i, _):
        for j in range(D // L):
            buf[i, pl.ds(j * L, L)] = jnp.zeros((L,), jnp.float32)
        return 0
    lax.fori_loop(0, nrows, zr, 0)


def _zero_shared(buf, sh, s, rows_per_sub):
    if rows_per_sub <= 64:
        pltpu.sync_copy(buf.at[pl.ds(0, rows_per_sub)],
                        sh.at[pl.ds(s * rows_per_sub, rows_per_sub)])
    else:
        def zc(i, _):
            pltpu.sync_copy(buf.at[pl.ds(0, 64)],
                            sh.at[pl.ds(s * rows_per_sub + i * 64, 64)])
            return 0
        lax.fori_loop(0, rows_per_sub // 64, zc, 0)


def _writeback(buf, sh, out, c, s, rows_per_sub):
    if rows_per_sub <= 64:
        pltpu.sync_copy(sh.at[pl.ds(s * rows_per_sub, rows_per_sub)],
                        buf.at[pl.ds(0, rows_per_sub)])
        pltpu.sync_copy(buf.at[pl.ds(0, rows_per_sub)],
                        out.at[c, pl.ds(s * rows_per_sub, rows_per_sub)])
    else:
        def wc(i, _):
            base = s * rows_per_sub + i * 64
            pltpu.sync_copy(sh.at[pl.ds(base, 64)], buf.at[pl.ds(0, 64)])
            pltpu.sync_copy(buf.at[pl.ds(0, 64)], out.at[c, pl.ds(base, 64)])
            return 0
        lax.fori_loop(0, rows_per_sub // 64, wc, 0)


@functools.partial(
    pl.kernel,
    out_type=(jax.ShapeDtypeStruct((NSp, D), jnp.float32),
              jax.ShapeDtypeStruct((NTp, D), jnp.float32),
              jax.ShapeDtypeStruct((NTp, D), jnp.float32)),
    mesh=_mesh,
    scratch_types=[pltpu.VMEM((1, 128), jnp.int32),
                   pltpu.VMEM((1, 128), jnp.int32),
                   pltpu.VMEM((128, D), jnp.float32),
                   pltpu.VMEM((128, D), jnp.float32),
                   pltpu.SemaphoreType.DMA,
                   pltpu.SemaphoreType.DMA],
)
def _k1(emb, w1, tuw, tiw, sids2d, stid2d, su2d, si2d,
        h_out, wu_out, wi_out, ia, ib, ra, rb, semA, semB):
    w = _wid()
    nch_h = NSp // 128

    def hchunk(t, _):
        ch = t * NW + w

        @pl.when(ch < nch_h)
        def _():
            pltpu.sync_copy(sids2d.at[pl.ds(ch, 1)], ia)
            pltpu.sync_copy(stid2d.at[pl.ds(ch, 1)], ib)
            pltpu.async_copy(emb.at[ia.at[0]], ra, semA)
            pltpu.async_copy(w1.at[ib.at[0]], rb, semB)
            pltpu.make_async_copy(emb.at[ia.at[0]], ra, semA).wait()
            pltpu.make_async_copy(w1.at[ib.at[0]], rb, semB).wait()

            def mrow(kk, _):
                for j in range(D // L):
                    ra[kk, pl.ds(j * L, L)] = (ra[kk, pl.ds(j * L, L)] *
                                               rb[kk, pl.ds(j * L, L)])
                return 0
            lax.fori_loop(0, 128, mrow, 0)
            pltpu.sync_copy(ra, h_out.at[pl.ds(ch * 128, 128)])
        return 0
    lax.fori_loop(0, nch_h // NW + 1, hchunk, 0)

    nch_t = NTp // 128

    def tchunk(t, _):
        ch = t * NW + w

        @pl.when(ch < nch_t)
        def _():
            pltpu.sync_copy(su2d.at[pl.ds(ch, 1)], ia)
            pltpu.sync_copy(si2d.at[pl.ds(ch, 1)], ib)
            pltpu.async_copy(tuw.at[ia.at[0]], ra, semA)
            pltpu.async_copy(tiw.at[ib.at[0]], rb, semB)
            pltpu.make_async_copy(tuw.at[ia.at[0]], ra, semA).wait()
            pltpu.sync_copy(ra, wu_out.at[pl.ds(ch * 128, 128)])
            pltpu.make_async_copy(tiw.at[ib.at[0]], rb, semB).wait()
            pltpu.sync_copy(rb, wi_out.at[pl.ds(ch * 128, 128)])
        return 0
    lax.fori_loop(0, nch_t // NW + 1, tchunk, 0)


def _make_passA(ndst, nchunks):
    cpw = nchunks // NW

    @functools.partial(
        pl.kernel,
        out_type=jax.ShapeDtypeStruct((NC, ndst, D), jnp.float32),
        mesh=_mesh,
        scratch_types=[pltpu.VMEM((1, 128), jnp.int32),
                       pltpu.VMEM((1, 128), jnp.int32),
                       pltpu.VMEM((1, 128), jnp.int32),
                       pltpu.VMEM((1, 128), jnp.int32),
                       pltpu.VMEM((128, D), jnp.float32),
                       pltpu.VMEM((128, D), jnp.float32),
                       pltpu.VMEM_SHARED((ndst, D), jnp.float32),
                       pltpu.SemaphoreType.DMA,
                       pltpu.SemaphoreType.DMA],
    )
    def kA(h, src2d, dst2d, accp,
           si0, di0, si1, di1, rows0, rows1, acc_sh, semA, semB):
        c, s = lax.axis_index("c"), lax.axis_index("s")
        w = s * NC + c
        _zero_rows(rows0, 64)
        _zero_shared(rows0, acc_sh, s, ndst // NSC)
        plsc.subcore_barrier()
        base = w * cpw
        pltpu.sync_copy(src2d.at[pl.ds(base, 1)], si0)
        pltpu.sync_copy(dst2d.at[pl.ds(base, 1)], di0)
        pltpu.async_copy(h.at[si0.at[0]], rows0, semA)

        def pair(t2, _):
            a = base + 2 * t2
            pltpu.sync_copy(src2d.at[pl.ds(a + 1, 1)], si1)
            pltpu.sync_copy(dst2d.at[pl.ds(a + 1, 1)], di1)
            pltpu.async_copy(h.at[si1.at[0]], rows1, semB)
            pltpu.make_async_copy(h.at[si0.at[0]], rows0, semA).wait()
            pltpu.sync_copy(rows0, acc_sh.at[di0.at[0]], add=True)

            @pl.when(t2 + 1 < cpw // 2)
            def _():
                pltpu.sync_copy(src2d.at[pl.ds(a + 2, 1)], si0)
                pltpu.sync_copy(dst2d.at[pl.ds(a + 2, 1)], di0)
                pltpu.async_copy(h.at[si0.at[0]], rows0, semA)
            pltpu.make_async_copy(h.at[si1.at[0]], rows1, semB).wait()
            pltpu.sync_copy(rows1, acc_sh.at[di1.at[0]], add=True)
            return 0
        lax.fori_loop(0, cpw // 2, pair, 0)
        plsc.subcore_barrier()
        _writeback(rows0, acc_sh, accp, c, s, ndst // NSC)

    return kA


def _make_combine(ndst):
    nch = ndst // 128

    @functools.partial(
        pl.kernel,
        out_type=jax.ShapeDtypeStruct((ndst, D), jnp.float32),
        mesh=_mesh,
        scratch_types=[pltpu.VMEM((128, D), jnp.float32),
                       pltpu.VMEM((128, D), jnp.float32)],
    )
    def kC(accp, out, ra, rb):
        w = _wid()

        def chunk(t, _):
            ch = t * NW + w

            @pl.when(ch < nch)
            def _():
                pltpu.sync_copy(accp.at[0, pl.ds(ch * 128, 128)], ra)
                pltpu.sync_copy(accp.at[1, pl.ds(ch * 128, 128)], rb)

                def arow(kk, _):
                    for j in range(D // L):
                        ra[kk, pl.ds(j * L, L)] = (ra[kk, pl.ds(j * L, L)] +
                                                   rb[kk, pl.ds(j * L, L)])
                    return 0
                lax.fori_loop(0, 128, arow, 0)
                pltpu.sync_copy(ra, out.at[pl.ds(ch * 128, 128)])
            return 0
        lax.fori_loop(0, nch // NW + 1, chunk, 0)

    return kC


def _make_passBC(ndst, nchunks, pipelined=False):
    cpw = nchunks // NW
    nz = ndst // 128
    nzs = 8
    nzp = nzs * NSC
    assert nz <= nzp
    nbuf = 2 if pipelined else 1

    @functools.partial(
        pl.kernel,
        out_type=(jax.ShapeDtypeStruct((NC, ndst, D), jnp.float32),
                  jax.ShapeDtypeStruct((NC, nzp, D), jnp.float32)),
        mesh=_mesh,
        scratch_types=[[pltpu.VMEM((1, 128), jnp.int32)] * nbuf,
                       [pltpu.VMEM((1, 128), jnp.int32)] * nbuf,
                       pltpu.VMEM((1, 128), jnp.int32),
                       pltpu.VMEM((8, L), jnp.int32),
                       pltpu.VMEM((8, L), jnp.float32),
                       [pltpu.VMEM((128, D), jnp.float32)] * nbuf,
                       [pltpu.VMEM((128, D), jnp.float32)] * nbuf,
                       pltpu.VMEM_SHARED((ndst, D), jnp.float32),
                       pltpu.VMEM_SHARED((nzp, D), jnp.float32),
                       [pltpu.SemaphoreType.DMA] * nbuf,
                       [pltpu.SemaphoreType.DMA] * nbuf],
    )
    def kBC(h, sumh, src2d, dst2d, outp, zp,
            sis, dis, zi, lo78, ebuf, rhs, rss, out_sh, z_sh, semAs, semBs):
        c, s = lax.axis_index("c"), lax.axis_index("s")
        w = s * NC + c
        rh0 = rhs[0]
        _zero_rows(rh0, 64)
        _zero_shared(rh0, out_sh, s, ndst // NSC)
        pltpu.sync_copy(rh0.at[pl.ds(0, nzs)], z_sh.at[pl.ds(s * nzs, nzs)])
        plsc.subcore_barrier()
        lanes = lax.broadcasted_iota(jnp.int32, (L,), 0)

        def load_and_fire(b, row):
            pltpu.sync_copy(src2d.at[pl.ds(row, 1)], sis[b])
            pltpu.sync_copy(dst2d.at[pl.ds(row, 1)], dis[b])
            pltpu.async_copy(h.at[sis[b].at[0]], rhs[b], semAs[b])
            pltpu.async_copy(sumh.at[dis[b].at[0]], rss[b], semBs[b])

        def process(b):
            si, di, rh, rs = sis[b], dis[b], rhs[b], rss[b]
            pltpu.make_async_copy(h.at[si.at[0]], rh, semAs[b]).wait()
            pltpu.make_async_copy(sumh.at[di.at[0]], rs, semBs[b]).wait()
            for j in range(128 // L):
                dv = di[0, pl.ds(j * L, L)]
                zi[0, pl.ds(j * L, L)] = jnp.right_shift(dv, 7)
                lo78[j, pl.ds(0, L)] = dv & 127

            def group(g, _):
                onehots = []
                for k2 in range(L):
                    kk = g * L + k2
                    acc0 = rh[kk, pl.ds(0, L)] * rs[kk, pl.ds(0, L)]
                    acc1 = rh[kk, pl.ds(4 * L, L)] * rs[kk, pl.ds(4 * L, L)]
                    for j in range(1, 4):
                        acc0 = acc0 + (rh[kk, pl.ds(j * L, L)] *
                                       rs[kk, pl.ds(j * L, L)])
                        acc1 = acc1 + (rh[kk, pl.ds((j + 4) * L, L)] *
                                       rs[kk, pl.ds((j + 4) * L, L)])
                    acc = acc0 + acc1
                    e = [acc[l] for l in range(L)]
                    while len(e) > 1:
                        e = [e[i] + e[i + 1] for i in range(0, len(e), 2)]
                    onehots.append(jnp.where(lanes == k2, e[0], 0.0))
                while len(onehots) > 1:
                    onehots = [onehots[i] + onehots[i + 1]
                               for i in range(0, len(onehots), 2)]
                ev = jnp.exp(onehots[0])
                ebuf[g, pl.ds(0, L)] = ev
                for k2 in range(L):
                    kk = g * L + k2
                    e = ev[k2]
                    for j in range(D // L):
                        rh[kk, pl.ds(j * L, L)] = rh[kk, pl.ds(j * L, L)] * e
                return 0
            lax.fori_loop(0, 128 // L, group, 0)
            pltpu.sync_copy(rh, out_sh.at[di.at[0]], add=True)

            def zgroup(g, _):
                ev = ebuf[g, pl.ds(0, L)]
                lv = lo78[g, pl.ds(0, L)]
                for k2 in range(L):
                    kk = g * L + k2
                    e = ev[k2]
                    low7 = lv[k2]
                    for j in range(D // L):
                        rh[kk, pl.ds(j * L, L)] = jnp.where(
                            lanes + (j * L) == low7, e, 0.0)
                return 0
            lax.fori_loop(0, 128 // L, zgroup, 0)
            pltpu.sync_copy(rh, z_sh.at[zi.at[0]], add=True)

        base = w * cpw
        if not pipelined:
            def step(t, _):
                load_and_fire(0, base + t)
                process(0)
                return 0
            lax.fori_loop(0, cpw, step, 0)
        else:
            load_and_fire(0, base)

            def pair(t2, _):
                a = base + 2 * t2
                load_and_fire(1, a + 1)
                process(0)

                @pl.when(t2 + 1 < cpw // 2)
                def _():
                    load_and_fire(0, a + 2)
                process(1)
                return 0
            lax.fori_loop(0, cpw // 2, pair, 0)
        plsc.subcore_barrier()
        _writeback(rh0, out_sh, outp, c, s, ndst // NSC)
        pltpu.sync_copy(z_sh.at[pl.ds(s * nzs, nzs)], rh0.at[pl.ds(0, nzs)])
        pltpu.sync_copy(rh0.at[pl.ds(0, nzs)], zp.at[c, pl.ds(s * nzs, nzs)])

    return kBC


_k2 = _make_passA(NTp, E1p // 128)
_k3 = _make_combine(NTp)
_k4 = _make_passBC(NTp, E1p // 128)
_c23 = _make_combine(NUp)


@functools.partial(
    pl.kernel,
    out_type=(jax.ShapeDtypeStruct((NC, NUp, D), jnp.float32),
              jax.ShapeDtypeStruct((NC, NIp, D), jnp.float32)),
    mesh=_mesh,
    scratch_types=[pltpu.VMEM((1, 128), jnp.int32),
                   pltpu.VMEM((1, 128), jnp.int32),
                   pltpu.VMEM((1, 128), jnp.int32),
                   pltpu.VMEM((1, 128), jnp.int32),
                   pltpu.VMEM((128, D), jnp.float32),
                   pltpu.VMEM((128, D), jnp.float32),
                   pltpu.VMEM_SHARED((NUp, D), jnp.float32),
                   pltpu.VMEM_SHARED((NIp, D), jnp.float32),
                   pltpu.SemaphoreType.DMA,
                   pltpu.SemaphoreType.DMA],
)
def _k6(hu, hi, src2_2d, dst2_2d, src3_2d, dst3_2d, accu, acci,
        si0, di0, si1, di1, rows0, rows1, accu_sh, acci_sh, semA, semB):
    c, s = lax.axis_index("c"), lax.axis_index("s")
    w = s * NC + c
    cpw = (E2p // 128) // NW
    _zero_rows(rows0, 64)
    _zero_shared(rows0, accu_sh, s, NUp // NSC)
    _zero_shared(rows0, acci_sh, s, NIp // NSC)
    plsc.subcore_barrier()

    def scan_edges(tbl, src2d, dst2d, sh):
        base = w * cpw
        pltpu.sync_copy(src2d.at[pl.ds(base, 1)], si0)
        pltpu.sync_copy(dst2d.at[pl.ds(base, 1)], di0)
        pltpu.async_copy(tbl.at[si0.at[0]], rows0, semA)

        def pair(t2, _):
            a = base + 2 * t2
            pltpu.sync_copy(src2d.at[pl.ds(a + 1, 1)], si1)
            pltpu.sync_copy(dst2d.at[pl.ds(a + 1, 1)], di1)
            pltpu.async_copy(tbl.at[si1.at[0]], rows1, semB)
            pltpu.make_async_copy(tbl.at[si0.at[0]], rows0, semA).wait()
            pltpu.sync_copy(rows0, sh.at[di0.at[0]], add=True)

            @pl.when(t2 + 1 < cpw // 2)
            def _():
                pltpu.sync_copy(src2d.at[pl.ds(a + 2, 1)], si0)
                pltpu.sync_copy(dst2d.at[pl.ds(a + 2, 1)], di0)
                pltpu.async_copy(tbl.at[si0.at[0]], rows0, semA)
            pltpu.make_async_copy(tbl.at[si1.at[0]], rows1, semB).wait()
            pltpu.sync_copy(rows1, sh.at[di1.at[0]], add=True)
            return 0
        lax.fori_loop(0, cpw // 2, pair, 0)

    scan_edges(hu, src2_2d, dst2_2d, accu_sh)
    scan_edges(hi, src3_2d, dst3_2d, acci_sh)
    plsc.subcore_barrier()
    _writeback(rows0, accu_sh, accu, c, s, NUp // NSC)
    _writeback(rows0, acci_sh, acci, c, s, NIp // NSC)


_k8u = _make_passBC(NUp, E2p // 128, pipelined=True)
_k8i = _make_passBC(NIp, E3p // 128, pipelined=True)


def _gelu_exact(x):
    return 0.5 * x * (1.0 + lax.erf(x * 0.7071067811865476))


def _k5_body(op_ref, zp_ref, w_ref, b_ref, wu_ref, wi_ref, hu_ref, hi_ref):
    t = op_ref[0] + op_ref[1]
    z = zp_ref[0] + zp_ref[1]
    tn = t / (z + 1e-9)
    tf = jnp.dot(tn, w_ref[...].T, preferred_element_type=jnp.float32) + b_ref[...]
    hu_ref[...] = _gelu_exact(tf * wu_ref[...])
    hi_ref[...] = _gelu_exact(tf * wi_ref[...])


def _k5(outp, zp2d, w, b2d, wu_rows, wi_rows):
    nblk = NTp // 128
    return pl.pallas_call(
        _k5_body,
        grid=(nblk,),
        in_specs=[
            pl.BlockSpec((NC, 128, D), lambda i: (0, i, 0)),
            pl.BlockSpec((NC, 128, 1), lambda i: (0, i, 0)),
            pl.BlockSpec((D, D), lambda i: (0, 0)),
            pl.BlockSpec((1, D), lambda i: (0, 0)),
            pl.BlockSpec((128, D), lambda i: (i, 0)),
            pl.BlockSpec((128, D), lambda i: (i, 0)),
        ],
        out_specs=[pl.BlockSpec((128, D), lambda i: (i, 0)),
                   pl.BlockSpec((128, D), lambda i: (i, 0))],
        out_shape=[jax.ShapeDtypeStruct((NTp, D), jnp.float32),
                   jax.ShapeDtypeStruct((NTp, D), jnp.float32)],
    )(outp, zp2d, w, b2d, wu_rows, wi_rows)


def _k9_body(up_ref, zu_ref, ip_ref, zi_ref, wu_ref, bu_ref, wi_ref, bi_ref,
             uf_ref, if_ref):
    u = (up_ref[0] + up_ref[1]) / (zu_ref[0] + zu_ref[1] + 1e-9)
    uf_ref[...] = jnp.dot(u, wu_ref[...].T,
                          preferred_element_type=jnp.float32) + bu_ref[...]
    v = (ip_ref[0] + ip_ref[1]) / (zi_ref[0] + zi_ref[1] + 1e-9)
    if_ref[...] = jnp.dot(v, wi_ref[...].T,
                          preferred_element_type=jnp.float32) + bi_ref[...]


def _k9(up, zu2d, ip, zi2d, u_w, u_b2d, i_w, i_b2d):
    nblk = NUp // 128
    return pl.pallas_call(
        _k9_body,
        grid=(nblk,),
        in_specs=[
            pl.BlockSpec((NC, 128, D), lambda i: (0, i, 0)),
            pl.BlockSpec((NC, 128, 1), lambda i: (0, i, 0)),
            pl.BlockSpec((NC, 128, D), lambda i: (0, i, 0)),
            pl.BlockSpec((NC, 128, 1), lambda i: (0, i, 0)),
            pl.BlockSpec((D, D), lambda i: (0, 0)),
            pl.BlockSpec((1, D), lambda i: (0, 0)),
            pl.BlockSpec((D, D), lambda i: (0, 0)),
            pl.BlockSpec((1, D), lambda i: (0, 0)),
        ],
        out_specs=[pl.BlockSpec((128, D), lambda i: (i, 0)),
                   pl.BlockSpec((128, D), lambda i: (i, 0))],
        out_shape=[jax.ShapeDtypeStruct((NUp, D), jnp.float32),
                   jax.ShapeDtypeStruct((NIp, D), jnp.float32)],
    )(up, zu2d, ip, zi2d, u_w, u_b2d, i_w, i_b2d)


def _pad_idx(x, n, mod):
    extra = n - x.shape[0]
    fill = jnp.arange(extra, dtype=jnp.int32) % mod
    return jnp.concatenate([x.astype(jnp.int32), fill])


def _pad_dst(x, n, real, padspace):
    extra = n - x.shape[0]
    fill = real + (jnp.arange(extra, dtype=jnp.int32) % padspace)
    return jnp.concatenate([x.astype(jnp.int32), fill])


def kernel(emb_table, sentence_w1, sent_lin_w, sent_lin_b, user_lin_w,
           user_lin_b, item_lin_w, item_lin_b, topic_user_w, topic_item_w,
           sentence_ids, stid_sent, src1, dst1, stid_user, src2, dst2,
           stid_item, src3, dst3):
    sids2d = _pad_idx(sentence_ids, NSp, VOCAB).reshape(-1, 128)
    stid2d = _pad_idx(stid_sent, NSp, 1024).reshape(-1, 128)
    su2d = _pad_idx(stid_user, NTp, 1024).reshape(-1, 128)
    si2d = _pad_idx(stid_item, NTp, 1024).reshape(-1, 128)
    src1_2d = _pad_idx(src1, E1p, NS).reshape(-1, 128)
    dst1_2d = _pad_dst(dst1, E1p, NT, NTp - NT).reshape(-1, 128)
    src2_2d = _pad_idx(src2, E2p, NT).reshape(-1, 128)
    dst2_2d = _pad_dst(dst2, E2p, NU, NUp - NU).reshape(-1, 128)
    src3_2d = _pad_idx(src3, E3p, NT).reshape(-1, 128)
    dst3_2d = _pad_dst(dst3, E3p, NI, NIp - NI).reshape(-1, 128)

    h, wu_rows, wi_rows = _k1(emb_table, sentence_w1, topic_user_w,
                              topic_item_w, sids2d, stid2d, su2d, si2d)

    sumh_p = _k2(h, src1_2d, dst1_2d)
    sumh = _k3(sumh_p)
    outp, zp = _k4(h, sumh, src1_2d, dst1_2d)
    zp2d = zp.reshape(NC, -1)[:, :NTp].reshape(NC, NTp, 1)
    hu, hi = _k5(outp, zp2d, sent_lin_w, sent_lin_b.reshape(1, D),
                 wu_rows, wi_rows)

    accu_p, acci_p = _k6(hu, hi, src2_2d, dst2_2d, src3_2d, dst3_2d)
    sumh2 = _c23(accu_p)
    sumh3 = _c23(acci_p)
    up, zu = _k8u(hu, sumh2, src2_2d, dst2_2d)
    ip, zi = _k8i(hi, sumh3, src3_2d, dst3_2d)
    zu2d = zu.reshape(NC, -1)[:, :NUp].reshape(NC, NUp, 1)
    zi2d = zi.reshape(NC, -1)[:, :NIp].reshape(NC, NIp, 1)
    user_feat, item_feat = _k9(up, zu2d, ip, zi2d,
                               user_lin_w, user_lin_b.reshape(1, D),
                               item_lin_w, item_lin_b.reshape(1, D))
    return (user_feat[:NU], item_feat[:NI])


# speedup vs baseline: 1.5816x; 1.0078x over previous
import functools
import jax
import jax.numpy as jnp
from jax import lax
from jax.experimental import pallas as pl
from jax.experimental.pallas import tpu as pltpu, tpu_sc as plsc

NS = 50000
NT = 10000
NU = 5000
NI = 5000
E1 = 500000
E2 = 320000
E3 = 320000
VOCAB = 200000
D = 128
L = 16
NC = 2
NSC = 16
NW = NC * NSC

NSp = 50176
NTp = 10240
NUp = 5120
NIp = 5120
E1p = 507904
E2p = 327680
E3p = 327680

_mesh = plsc.VectorSubcoreMesh(core_axis_name="c", subcore_axis_name="s")


def _wid():
    return lax.axis_index("s") * NC + lax.axis_index("c")


def _zero_rows(buf, nrows):
    def zr(i, _):
        for j in range(D // L):
            buf[i, pl.ds(j * L, L)] = jnp.zeros((L,), jnp.float32)
        return 0
    lax.fori_loop(0, nrows, zr, 0)


def _zero_shared(buf, sh, s, rows_per_sub):
    if rows_per_sub <= 64:
        pltpu.sync_copy(buf.at[pl.ds(0, rows_per_sub)],
                        sh.at[pl.ds(s * rows_per_sub, rows_per_sub)])
    else:
        def zc(i, _):
            pltpu.sync_copy(buf.at[pl.ds(0, 64)],
                            sh.at[pl.ds(s * rows_per_sub + i * 64, 64)])
            return 0
        lax.fori_loop(0, rows_per_sub // 64, zc, 0)


def _writeback(buf, sh, out, c, s, rows_per_sub):
    if rows_per_sub <= 64:
        pltpu.sync_copy(sh.at[pl.ds(s * rows_per_sub, rows_per_sub)],
                        buf.at[pl.ds(0, rows_per_sub)])
        pltpu.sync_copy(buf.at[pl.ds(0, rows_per_sub)],
                        out.at[c, pl.ds(s * rows_per_sub, rows_per_sub)])
    else:
        def wc(i, _):
            base = s * rows_per_sub + i * 64
            pltpu.sync_copy(sh.at[pl.ds(base, 64)], buf.at[pl.ds(0, 64)])
            pltpu.sync_copy(buf.at[pl.ds(0, 64)], out.at[c, pl.ds(base, 64)])
            return 0
        lax.fori_loop(0, rows_per_sub // 64, wc, 0)


@functools.partial(
    pl.kernel,
    out_type=(jax.ShapeDtypeStruct((NSp, D), jnp.float32),
              jax.ShapeDtypeStruct((NTp, D), jnp.float32),
              jax.ShapeDtypeStruct((NTp, D), jnp.float32)),
    mesh=_mesh,
    scratch_types=[pltpu.VMEM((1, 128), jnp.int32),
                   pltpu.VMEM((1, 128), jnp.int32),
                   pltpu.VMEM((128, D), jnp.float32),
                   pltpu.VMEM((128, D), jnp.float32),
                   pltpu.SemaphoreType.DMA,
                   pltpu.SemaphoreType.DMA],
)
def _k1(emb, w1, tuw, tiw, sids2d, stid2d, su2d, si2d,
        h_out, wu_out, wi_out, ia, ib, ra, rb, semA, semB):
    w = _wid()
    nch_h = NSp // 128

    def hchunk(t, _):
        ch = t * NW + w

        @pl.when(ch < nch_h)
        def _():
            pltpu.sync_copy(sids2d.at[pl.ds(ch, 1)], ia)
            pltpu.sync_copy(stid2d.at[pl.ds(ch, 1)], ib)
            pltpu.async_copy(emb.at[ia.at[0]], ra, semA)
            pltpu.async_copy(w1.at[ib.at[0]], rb, semB)
            pltpu.make_async_copy(emb.at[ia.at[0]], ra, semA).wait()
            pltpu.make_async_copy(w1.at[ib.at[0]], rb, semB).wait()

            def mrow(kk, _):
                for j in range(D // L):
                    ra[kk, pl.ds(j * L, L)] = (ra[kk, pl.ds(j * L, L)] *
                                               rb[kk, pl.ds(j * L, L)])
                return 0
            lax.fori_loop(0, 128, mrow, 0)
            pltpu.sync_copy(ra, h_out.at[pl.ds(ch * 128, 128)])
        return 0
    lax.fori_loop(0, nch_h // NW + 1, hchunk, 0)

    nch_t = NTp // 128

    def tchunk(t, _):
        ch = t * NW + w

        @pl.when(ch < nch_t)
        def _():
            pltpu.sync_copy(su2d.at[pl.ds(ch, 1)], ia)
            pltpu.sync_copy(si2d.at[pl.ds(ch, 1)], ib)
            pltpu.async_copy(tuw.at[ia.at[0]], ra, semA)
            pltpu.async_copy(tiw.at[ib.at[0]], rb, semB)
            pltpu.make_async_copy(tuw.at[ia.at[0]], ra, semA).wait()
            pltpu.sync_copy(ra, wu_out.at[pl.ds(ch * 128, 128)])
            pltpu.make_async_copy(tiw.at[ib.at[0]], rb, semB).wait()
            pltpu.sync_copy(rb, wi_out.at[pl.ds(ch * 128, 128)])
        return 0
    lax.fori_loop(0, nch_t // NW + 1, tchunk, 0)


def _make_passA(ndst, nchunks):
    cpw = nchunks // NW

    @functools.partial(
        pl.kernel,
        out_type=jax.ShapeDtypeStruct((NC, ndst, D), jnp.float32),
        mesh=_mesh,
        scratch_types=[pltpu.VMEM((1, 128), jnp.int32),
                       pltpu.VMEM((1, 128), jnp.int32),
                       pltpu.VMEM((1, 128), jnp.int32),
                       pltpu.VMEM((1, 128), jnp.int32),
                       pltpu.VMEM((128, D), jnp.float32),
                       pltpu.VMEM((128, D), jnp.float32),
                       pltpu.VMEM_SHARED((ndst, D), jnp.float32),
                       pltpu.SemaphoreType.DMA,
                       pltpu.SemaphoreType.DMA],
    )
    def kA(h, src2d, dst2d, accp,
           si0, di0, si1, di1, rows0, rows1, acc_sh, semA, semB):
        c, s = lax.axis_index("c"), lax.axis_index("s")
        w = s * NC + c
        _zero_rows(rows0, 64)
        _zero_shared(rows0, acc_sh, s, ndst // NSC)
        plsc.subcore_barrier()
        base = w * cpw
        pltpu.sync_copy(src2d.at[pl.ds(base, 1)], si0)
        pltpu.sync_copy(dst2d.at[pl.ds(base, 1)], di0)
        pltpu.async_copy(h.at[si0.at[0]], rows0, semA)

        def pair(t2, _):
            a = base + 2 * t2
            pltpu.sync_copy(src2d.at[pl.ds(a + 1, 1)], si1)
            pltpu.sync_copy(dst2d.at[pl.ds(a + 1, 1)], di1)
            pltpu.async_copy(h.at[si1.at[0]], rows1, semB)
            pltpu.make_async_copy(h.at[si0.at[0]], rows0, semA).wait()
            pltpu.sync_copy(rows0, acc_sh.at[di0.at[0]], add=True)

            @pl.when(t2 + 1 < cpw // 2)
            def _():
                pltpu.sync_copy(src2d.at[pl.ds(a + 2, 1)], si0)
                pltpu.sync_copy(dst2d.at[pl.ds(a + 2, 1)], di0)
                pltpu.async_copy(h.at[si0.at[0]], rows0, semA)
            pltpu.make_async_copy(h.at[si1.at[0]], rows1, semB).wait()
            pltpu.sync_copy(rows1, acc_sh.at[di1.at[0]], add=True)
            return 0
        lax.fori_loop(0, cpw // 2, pair, 0)
        plsc.subcore_barrier()
        _writeback(rows0, acc_sh, accp, c, s, ndst // NSC)

    return kA


def _make_combine(ndst):
    nch = ndst // 128

    @functools.partial(
        pl.kernel,
        out_type=jax.ShapeDtypeStruct((ndst, D), jnp.float32),
        mesh=_mesh,
        scratch_types=[pltpu.VMEM((128, D), jnp.float32),
                       pltpu.VMEM((128, D), jnp.float32)],
    )
    def kC(accp, out, ra, rb):
        w = _wid()

        def chunk(t, _):
            ch = t * NW + w

            @pl.when(ch < nch)
            def _():
                pltpu.sync_copy(accp.at[0, pl.ds(ch * 128, 128)], ra)
                pltpu.sync_copy(accp.at[1, pl.ds(ch * 128, 128)], rb)

                def arow(kk, _):
                    for j in range(D // L):
                        ra[kk, pl.ds(j * L, L)] = (ra[kk, pl.ds(j * L, L)] +
                                                   rb[kk, pl.ds(j * L, L)])
                    return 0
                lax.fori_loop(0, 128, arow, 0)
                pltpu.sync_copy(ra, out.at[pl.ds(ch * 128, 128)])
            return 0
        lax.fori_loop(0, nch // NW + 1, chunk, 0)

    return kC


def _make_passBC(ndst, nchunks, pipelined=False):
    cpw = nchunks // NW
    nz = ndst // 128
    nzs = 8
    nzp = nzs * NSC
    assert nz <= nzp
    nbuf = 2 if pipelined else 1

    @functools.partial(
        pl.kernel,
        out_type=(jax.ShapeDtypeStruct((NC, ndst, D), jnp.float32),
                  jax.ShapeDtypeStruct((NC, nzp, D), jnp.float32)),
        mesh=_mesh,
        scratch_types=[[pltpu.VMEM((1, 128), jnp.int32)] * nbuf,
                       [pltpu.VMEM((1, 128), jnp.int32)] * nbuf,
                       pltpu.VMEM((1, 128), jnp.int32),
                       pltpu.VMEM((8, L), jnp.int32),
                       pltpu.VMEM((8, L), jnp.float32),
                       [pltpu.VMEM((128, D), jnp.float32)] * nbuf,
                       [pltpu.VMEM((128, D), jnp.float32)] * nbuf,
                       pltpu.VMEM_SHARED((ndst, D), jnp.float32),
                       pltpu.VMEM_SHARED((nzp, D), jnp.float32),
                       [pltpu.SemaphoreType.DMA] * nbuf,
                       [pltpu.SemaphoreType.DMA] * nbuf],
    )
    def kBC(h, sumh, src2d, dst2d, outp, zp,
            sis, dis, zi, lo78, ebuf, rhs, rss, out_sh, z_sh, semAs, semBs):
        c, s = lax.axis_index("c"), lax.axis_index("s")
        w = s * NC + c
        rh0 = rhs[0]
        _zero_rows(rh0, 64)
        _zero_shared(rh0, out_sh, s, ndst // NSC)
        pltpu.sync_copy(rh0.at[pl.ds(0, nzs)], z_sh.at[pl.ds(s * nzs, nzs)])
        plsc.subcore_barrier()
        lanes = lax.broadcasted_iota(jnp.int32, (L,), 0)

        def load_and_fire(b, row):
            pltpu.sync_copy(src2d.at[pl.ds(row, 1)], sis[b])
            pltpu.sync_copy(dst2d.at[pl.ds(row, 1)], dis[b])
            pltpu.async_copy(h.at[sis[b].at[0]], rhs[b], semAs[b])
            pltpu.async_copy(sumh.at[dis[b].at[0]], rss[b], semBs[b])

        def process(b):
            si, di, rh, rs = sis[b], dis[b], rhs[b], rss[b]
            pltpu.make_async_copy(h.at[si.at[0]], rh, semAs[b]).wait()
            pltpu.make_async_copy(sumh.at[di.at[0]], rs, semBs[b]).wait()
            for j in range(128 // L):
                dv = di[0, pl.ds(j * L, L)]
                zi[0, pl.ds(j * L, L)] = jnp.right_shift(dv, 7)
                lo78[j, pl.ds(0, L)] = dv & 127

            def group(g, _):
                onehots = []
                for k2 in range(L):
                    kk = g * L + k2
                    acc0 = rh[kk, pl.ds(0, L)] * rs[kk, pl.ds(0, L)]
                    acc1 = rh[kk, pl.ds(4 * L, L)] * rs[kk, pl.ds(4 * L, L)]
                    for j in range(1, 4):
                        acc0 = acc0 + (rh[kk, pl.ds(j * L, L)] *
                                       rs[kk, pl.ds(j * L, L)])
                        acc1 = acc1 + (rh[kk, pl.ds((j + 4) * L, L)] *
                                       rs[kk, pl.ds((j + 4) * L, L)])
                    acc = acc0 + acc1
                    e = [acc[l] for l in range(L)]
                    while len(e) > 1:
                        e = [e[i] + e[i + 1] for i in range(0, len(e), 2)]
                    onehots.append(jnp.where(lanes == k2, e[0], 0.0))
                while len(onehots) > 1:
                    onehots = [onehots[i] + onehots[i + 1]
                               for i in range(0, len(onehots), 2)]
                ev = jnp.exp(onehots[0])
                ebuf[g, pl.ds(0, L)] = ev
                for k2 in range(L):
                    kk = g * L + k2
                    e = ev[k2]
                    for j in range(D // L):
                        rh[kk, pl.ds(j * L, L)] = rh[kk, pl.ds(j * L, L)] * e
                return 0
            lax.fori_loop(0, 128 // L, group, 0)

            def zgroup(g, _):
                ev = ebuf[g, pl.ds(0, L)]
                lv = lo78[g, pl.ds(0, L)]
                for k2 in range(L):
                    kk = g * L + k2
                    e = ev[k2]
                    low7 = lv[k2]
                    for j in range(D // L):
                        rs[kk, pl.ds(j * L, L)] = jnp.where(
                            lanes + (j * L) == low7, e, 0.0)
                return 0
            lax.fori_loop(0, 128 // L, zgroup, 0)
            pltpu.async_copy(rh, out_sh.at[di.at[0]], semAs[b], add=True)
            pltpu.async_copy(rs, z_sh.at[zi.at[0]], semBs[b], add=True)
            pltpu.make_async_copy(rh, out_sh.at[di.at[0]], semAs[b]).wait()
            pltpu.make_async_copy(rs, z_sh.at[zi.at[0]], semBs[b]).wait()

        base = w * cpw
        if not pipelined:
            def step(t, _):
                load_and_fire(0, base + t)
                process(0)
                return 0
            lax.fori_loop(0, cpw, step, 0)
        else:
            load_and_fire(0, base)

            def pair(t2, _):
                a = base + 2 * t2
                load_and_fire(1, a + 1)
                process(0)

                @pl.when(t2 + 1 < cpw // 2)
                def _():
                    load_and_fire(0, a + 2)
                process(1)
                return 0
            lax.fori_loop(0, cpw // 2, pair, 0)
        plsc.subcore_barrier()
        _writeback(rh0, out_sh, outp, c, s, ndst // NSC)
        pltpu.sync_copy(z_sh.at[pl.ds(s * nzs, nzs)], rh0.at[pl.ds(0, nzs)])
        pltpu.sync_copy(rh0.at[pl.ds(0, nzs)], zp.at[c, pl.ds(s * nzs, nzs)])

    return kBC


_k2 = _make_passA(NTp, E1p // 128)
_k3 = _make_combine(NTp)
_k4 = _make_passBC(NTp, E1p // 128)
_c23 = _make_combine(NUp)


@functools.partial(
    pl.kernel,
    out_type=(jax.ShapeDtypeStruct((NC, NUp, D), jnp.float32),
              jax.ShapeDtypeStruct((NC, NIp, D), jnp.float32)),
    mesh=_mesh,
    scratch_types=[pltpu.VMEM((1, 128), jnp.int32),
                   pltpu.VMEM((1, 128), jnp.int32),
                   pltpu.VMEM((1, 128), jnp.int32),
                   pltpu.VMEM((1, 128), jnp.int32),
                   pltpu.VMEM((128, D), jnp.float32),
                   pltpu.VMEM((128, D), jnp.float32),
                   pltpu.VMEM_SHARED((NUp, D), jnp.float32),
                   pltpu.VMEM_SHARED((NIp, D), jnp.float32),
                   pltpu.SemaphoreType.DMA,
                   pltpu.SemaphoreType.DMA],
)
def _k6(hu, hi, src2_2d, dst2_2d, src3_2d, dst3_2d, accu, acci,
        si0, di0, si1, di1, rows0, rows1, accu_sh, acci_sh, semA, semB):
    c, s = lax.axis_index("c"), lax.axis_index("s")
    w = s * NC + c
    cpw = (E2p // 128) // NW
    _zero_rows(rows0, 64)
    _zero_shared(rows0, accu_sh, s, NUp // NSC)
    _zero_shared(rows0, acci_sh, s, NIp // NSC)
    plsc.subcore_barrier()

    def scan_edges(tbl, src2d, dst2d, sh):
        base = w * cpw
        pltpu.sync_copy(src2d.at[pl.ds(base, 1)], si0)
        pltpu.sync_copy(dst2d.at[pl.ds(base, 1)], di0)
        pltpu.async_copy(tbl.at[si0.at[0]], rows0, semA)

        def pair(t2, _):
            a = base + 2 * t2
            pltpu.sync_copy(src2d.at[pl.ds(a + 1, 1)], si1)
            pltpu.sync_copy(dst2d.at[pl.ds(a + 1, 1)], di1)
            pltpu.async_copy(tbl.at[si1.at[0]], rows1, semB)
            pltpu.make_async_copy(tbl.at[si0.at[0]], rows0, semA).wait()
            pltpu.sync_copy(rows0, sh.at[di0.at[0]], add=True)

            @pl.when(t2 + 1 < cpw // 2)
            def _():
                pltpu.sync_copy(src2d.at[pl.ds(a + 2, 1)], si0)
                pltpu.sync_copy(dst2d.at[pl.ds(a + 2, 1)], di0)
                pltpu.async_copy(tbl.at[si0.at[0]], rows0, semA)
            pltpu.make_async_copy(tbl.at[si1.at[0]], rows1, semB).wait()
            pltpu.sync_copy(rows1, sh.at[di1.at[0]], add=True)
            return 0
        lax.fori_loop(0, cpw // 2, pair, 0)

    scan_edges(hu, src2_2d, dst2_2d, accu_sh)
    scan_edges(hi, src3_2d, dst3_2d, acci_sh)
    plsc.subcore_barrier()
    _writeback(rows0, accu_sh, accu, c, s, NUp // NSC)
    _writeback(rows0, acci_sh, acci, c, s, NIp // NSC)


_k8u = _make_passBC(NUp, E2p // 128, pipelined=True)
_k8i = _make_passBC(NIp, E3p // 128, pipelined=True)


def _gelu_exact(x):
    return 0.5 * x * (1.0 + lax.erf(x * 0.7071067811865476))


def _k5_body(op_ref, zp_ref, w_ref, b_ref, wu_ref, wi_ref, hu_ref, hi_ref):
    t = op_ref[0] + op_ref[1]
    z = zp_ref[0] + zp_ref[1]
    tn = t / (z + 1e-9)
    tf = jnp.dot(tn, w_ref[...].T, preferred_element_type=jnp.float32) + b_ref[...]
    hu_ref[...] = _gelu_exact(tf * wu_ref[...])
    hi_ref[...] = _gelu_exact(tf * wi_ref[...])


def _k5(outp, zp2d, w, b2d, wu_rows, wi_rows):
    nblk = NTp // 128
    return pl.pallas_call(
        _k5_body,
        grid=(nblk,),
        in_specs=[
            pl.BlockSpec((NC, 128, D), lambda i: (0, i, 0)),
            pl.BlockSpec((NC, 128, 1), lambda i: (0, i, 0)),
            pl.BlockSpec((D, D), lambda i: (0, 0)),
            pl.BlockSpec((1, D), lambda i: (0, 0)),
            pl.BlockSpec((128, D), lambda i: (i, 0)),
            pl.BlockSpec((128, D), lambda i: (i, 0)),
        ],
        out_specs=[pl.BlockSpec((128, D), lambda i: (i, 0)),
                   pl.BlockSpec((128, D), lambda i: (i, 0))],
        out_shape=[jax.ShapeDtypeStruct((NTp, D), jnp.float32),
                   jax.ShapeDtypeStruct((NTp, D), jnp.float32)],
    )(outp, zp2d, w, b2d, wu_rows, wi_rows)


def _k9_body(up_ref, zu_ref, ip_ref, zi_ref, wu_ref, bu_ref, wi_ref, bi_ref,
             uf_ref, if_ref):
    u = (up_ref[0] + up_ref[1]) / (zu_ref[0] + zu_ref[1] + 1e-9)
    uf_ref[...] = jnp.dot(u, wu_ref[...].T,
                          preferred_element_type=jnp.float32) + bu_ref[...]
    v = (ip_ref[0] + ip_ref[1]) / (zi_ref[0] + zi_ref[1] + 1e-9)
    if_ref[...] = jnp.dot(v, wi_ref[...].T,
                          preferred_element_type=jnp.float32) + bi_ref[...]


def _k9(up, zu2d, ip, zi2d, u_w, u_b2d, i_w, i_b2d):
    nblk = NUp // 128
    return pl.pallas_call(
        _k9_body,
        grid=(nblk,),
        in_specs=[
            pl.BlockSpec((NC, 128, D), lambda i: (0, i, 0)),
            pl.BlockSpec((NC, 128, 1), lambda i: (0, i, 0)),
            pl.BlockSpec((NC, 128, D), lambda i: (0, i, 0)),
            pl.BlockSpec((NC, 128, 1), lambda i: (0, i, 0)),
            pl.BlockSpec((D, D), lambda i: (0, 0)),
            pl.BlockSpec((1, D), lambda i: (0, 0)),
            pl.BlockSpec((D, D), lambda i: (0, 0)),
            pl.BlockSpec((1, D), lambda i: (0, 0)),
        ],
        out_specs=[pl.BlockSpec((128, D), lambda i: (i, 0)),
                   pl.BlockSpec((128, D), lambda i: (i, 0))],
        out_shape=[jax.ShapeDtypeStruct((NUp, D), jnp.float32),
                   jax.ShapeDtypeStruct((NIp, D), jnp.float32)],
    )(up, zu2d, ip, zi2d, u_w, u_b2d, i_w, i_b2d)


def _pad_idx(x, n, mod):
    extra = n - x.shape[0]
    fill = jnp.arange(extra, dtype=jnp.int32) % mod
    return jnp.concatenate([x.astype(jnp.int32), fill])


def _pad_dst(x, n, real, padspace):
    extra = n - x.shape[0]
    fill = real + (jnp.arange(extra, dtype=jnp.int32) % padspace)
    return jnp.concatenate([x.astype(jnp.int32), fill])


def kernel(emb_table, sentence_w1, sent_lin_w, sent_lin_b, user_lin_w,
           user_lin_b, item_lin_w, item_lin_b, topic_user_w, topic_item_w,
           sentence_ids, stid_sent, src1, dst1, stid_user, src2, dst2,
           stid_item, src3, dst3):
    sids2d = _pad_idx(sentence_ids, NSp, VOCAB).reshape(-1, 128)
    stid2d = _pad_idx(stid_sent, NSp, 1024).reshape(-1, 128)
    su2d = _pad_idx(stid_user, NTp, 1024).reshape(-1, 128)
    si2d = _pad_idx(stid_item, NTp, 1024).reshape(-1, 128)
    src1_2d = _pad_idx(src1, E1p, NS).reshape(-1, 128)
    dst1_2d = _pad_dst(dst1, E1p, NT, NTp - NT).reshape(-1, 128)
    src2_2d = _pad_idx(src2, E2p, NT).reshape(-1, 128)
    dst2_2d = _pad_dst(dst2, E2p, NU, NUp - NU).reshape(-1, 128)
    src3_2d = _pad_idx(src3, E3p, NT).reshape(-1, 128)
    dst3_2d = _pad_dst(dst3, E3p, NI, NIp - NI).reshape(-1, 128)

    h, wu_rows, wi_rows = _k1(emb_table, sentence_w1, topic_user_w,
                              topic_item_w, sids2d, stid2d, su2d, si2d)

    sumh_p = _k2(h, src1_2d, dst1_2d)
    sumh = _k3(sumh_p)
    outp, zp = _k4(h, sumh, src1_2d, dst1_2d)
    zp2d = zp.reshape(NC, -1)[:, :NTp].reshape(NC, NTp, 1)
    hu, hi = _k5(outp, zp2d, sent_lin_w, sent_lin_b.reshape(1, D),
                 wu_rows, wi_rows)

    accu_p, acci_p = _k6(hu, hi, src2_2d, dst2_2d, src3_2d, dst3_2d)
    sumh2 = _c23(accu_p)
    sumh3 = _c23(acci_p)
    up, zu = _k8u(hu, sumh2, src2_2d, dst2_2d)
    ip, zi = _k8i(hi, sumh3, src3_2d, dst3_2d)
    zu2d = zu.reshape(NC, -1)[:, :NUp].reshape(NC, NUp, 1)
    zi2d = zi.reshape(NC, -1)[:, :NIp].reshape(NC, NIp, 1)
    user_feat, item_feat = _k9(up, zu2d, ip, zi2d,
                               user_lin_w, user_lin_b.reshape(1, D),
                               item_lin_w, item_lin_b.reshape(1, D))
    return (user_feat[:NU], item_feat[:NI])


# speedup vs baseline: 1.5993x; 1.0112x over previous
import functools
import jax
import jax.numpy as jnp
from jax import lax
from jax.experimental import pallas as pl
from jax.experimental.pallas import tpu as pltpu, tpu_sc as plsc

NS = 50000
NT = 10000
NU = 5000
NI = 5000
E1 = 500000
E2 = 320000
E3 = 320000
VOCAB = 200000
D = 128
L = 16
NC = 2
NSC = 16
NW = NC * NSC

NSp = 50176
NTp = 10240
NUp = 5120
NIp = 5120
E1p = 507904
E2p = 327680
E3p = 327680

_mesh = plsc.VectorSubcoreMesh(core_axis_name="c", subcore_axis_name="s")


def _wid():
    return lax.axis_index("s") * NC + lax.axis_index("c")


def _zero_rows(buf, nrows):
    def zr(i, _):
        for j in range(D // L):
            buf[i, pl.ds(j * L, L)] = jnp.zeros((L,), jnp.float32)
        return 0
    lax.fori_loop(0, nrows, zr, 0)


def _zero_shared(buf, sh, s, rows_per_sub):
    if rows_per_sub <= 64:
        pltpu.sync_copy(buf.at[pl.ds(0, rows_per_sub)],
                        sh.at[pl.ds(s * rows_per_sub, rows_per_sub)])
    else:
        def zc(i, _):
            pltpu.sync_copy(buf.at[pl.ds(0, 64)],
                            sh.at[pl.ds(s * rows_per_sub + i * 64, 64)])
            return 0
        lax.fori_loop(0, rows_per_sub // 64, zc, 0)


def _writeback(buf, sh, out, c, s, rows_per_sub):
    if rows_per_sub <= 64:
        pltpu.sync_copy(sh.at[pl.ds(s * rows_per_sub, rows_per_sub)],
                        buf.at[pl.ds(0, rows_per_sub)])
        pltpu.sync_copy(buf.at[pl.ds(0, rows_per_sub)],
                        out.at[c, pl.ds(s * rows_per_sub, rows_per_sub)])
    else:
        def wc(i, _):
            base = s * rows_per_sub + i * 64
            pltpu.sync_copy(sh.at[pl.ds(base, 64)], buf.at[pl.ds(0, 64)])
            pltpu.sync_copy(buf.at[pl.ds(0, 64)], out.at[c, pl.ds(base, 64)])
            return 0
        lax.fori_loop(0, rows_per_sub // 64, wc, 0)


@functools.partial(
    pl.kernel,
    out_type=(jax.ShapeDtypeStruct((NSp, D), jnp.float32),
              jax.ShapeDtypeStruct((NTp, D), jnp.float32),
              jax.ShapeDtypeStruct((NTp, D), jnp.float32)),
    mesh=_mesh,
    scratch_types=[pltpu.VMEM((1, 128), jnp.int32),
                   pltpu.VMEM((1, 128), jnp.int32),
                   pltpu.VMEM((128, D), jnp.float32),
                   pltpu.VMEM((128, D), jnp.float32),
                   pltpu.SemaphoreType.DMA,
                   pltpu.SemaphoreType.DMA],
)
def _k1(emb, w1, tuw, tiw, sids2d, stid2d, su2d, si2d,
        h_out, wu_out, wi_out, ia, ib, ra, rb, semA, semB):
    w = _wid()
    nch_h = NSp // 128

    def hchunk(t, _):
        ch = t * NW + w

        @pl.when(ch < nch_h)
        def _():
            pltpu.sync_copy(sids2d.at[pl.ds(ch, 1)], ia)
            pltpu.sync_copy(stid2d.at[pl.ds(ch, 1)], ib)
            pltpu.async_copy(emb.at[ia.at[0]], ra, semA)
            pltpu.async_copy(w1.at[ib.at[0]], rb, semB)
            pltpu.make_async_copy(emb.at[ia.at[0]], ra, semA).wait()
            pltpu.make_async_copy(w1.at[ib.at[0]], rb, semB).wait()

            def mrow(kk, _):
                for j in range(D // L):
                    ra[kk, pl.ds(j * L, L)] = (ra[kk, pl.ds(j * L, L)] *
                                               rb[kk, pl.ds(j * L, L)])
                return 0
            lax.fori_loop(0, 128, mrow, 0)
            pltpu.sync_copy(ra, h_out.at[pl.ds(ch * 128, 128)])
        return 0
    lax.fori_loop(0, nch_h // NW + 1, hchunk, 0)

    nch_t = NTp // 128

    def tchunk(t, _):
        ch = t * NW + w

        @pl.when(ch < nch_t)
        def _():
            pltpu.sync_copy(su2d.at[pl.ds(ch, 1)], ia)
            pltpu.sync_copy(si2d.at[pl.ds(ch, 1)], ib)
            pltpu.async_copy(tuw.at[ia.at[0]], ra, semA)
            pltpu.async_copy(tiw.at[ib.at[0]], rb, semB)
            pltpu.make_async_copy(tuw.at[ia.at[0]], ra, semA).wait()
            pltpu.sync_copy(ra, wu_out.at[pl.ds(ch * 128, 128)])
            pltpu.make_async_copy(tiw.at[ib.at[0]], rb, semB).wait()
            pltpu.sync_copy(rb, wi_out.at[pl.ds(ch * 128, 128)])
        return 0
    lax.fori_loop(0, nch_t // NW + 1, tchunk, 0)


def _make_passA(ndst, nchunks):
    cpw = nchunks // NW

    @functools.partial(
        pl.kernel,
        out_type=jax.ShapeDtypeStruct((NC, ndst, D), jnp.float32),
        mesh=_mesh,
        scratch_types=[pltpu.VMEM((1, 128), jnp.int32),
                       pltpu.VMEM((1, 128), jnp.int32),
                       pltpu.VMEM((1, 128), jnp.int32),
                       pltpu.VMEM((1, 128), jnp.int32),
                       pltpu.VMEM((128, D), jnp.float32),
                       pltpu.VMEM((128, D), jnp.float32),
                       pltpu.VMEM_SHARED((ndst, D), jnp.float32),
                       pltpu.SemaphoreType.DMA,
                       pltpu.SemaphoreType.DMA],
    )
    def kA(h, src2d, dst2d, accp,
           si0, di0, si1, di1, rows0, rows1, acc_sh, semA, semB):
        c, s = lax.axis_index("c"), lax.axis_index("s")
        w = s * NC + c
        _zero_rows(rows0, 64)
        _zero_shared(rows0, acc_sh, s, ndst // NSC)
        plsc.subcore_barrier()
        base = w * cpw
        pltpu.sync_copy(src2d.at[pl.ds(base, 1)], si0)
        pltpu.sync_copy(dst2d.at[pl.ds(base, 1)], di0)
        pltpu.async_copy(h.at[si0.at[0]], rows0, semA)

        def pair(t2, _):
            a = base + 2 * t2
            pltpu.sync_copy(src2d.at[pl.ds(a + 1, 1)], si1)
            pltpu.sync_copy(dst2d.at[pl.ds(a + 1, 1)], di1)
            pltpu.async_copy(h.at[si1.at[0]], rows1, semB)
            pltpu.make_async_copy(h.at[si0.at[0]], rows0, semA).wait()
            pltpu.sync_copy(rows0, acc_sh.at[di0.at[0]], add=True)

            @pl.when(t2 + 1 < cpw // 2)
            def _():
                pltpu.sync_copy(src2d.at[pl.ds(a + 2, 1)], si0)
                pltpu.sync_copy(dst2d.at[pl.ds(a + 2, 1)], di0)
                pltpu.async_copy(h.at[si0.at[0]], rows0, semA)
            pltpu.make_async_copy(h.at[si1.at[0]], rows1, semB).wait()
            pltpu.sync_copy(rows1, acc_sh.at[di1.at[0]], add=True)
            return 0
        lax.fori_loop(0, cpw // 2, pair, 0)
        plsc.subcore_barrier()
        _writeback(rows0, acc_sh, accp, c, s, ndst // NSC)

    return kA


def _make_combine(ndst):
    nblk = ndst // 512

    def body(p_ref, o_ref):
        o_ref[...] = p_ref[0] + p_ref[1]

    def kC(accp):
        return pl.pallas_call(
            body,
            grid=(nblk,),
            in_specs=[pl.BlockSpec((NC, 512, D), lambda i: (0, i, 0))],
            out_specs=pl.BlockSpec((512, D), lambda i: (i, 0)),
            out_shape=jax.ShapeDtypeStruct((ndst, D), jnp.float32),
        )(accp)

    return kC


def _make_passBC(ndst, nchunks, pipelined=False):
    cpw = nchunks // NW
    nz = ndst // 128
    nzs = 8
    nzp = nzs * NSC
    assert nz <= nzp
    nbuf = 2 if pipelined else 1

    @functools.partial(
        pl.kernel,
        out_type=(jax.ShapeDtypeStruct((NC, ndst, D), jnp.float32),
                  jax.ShapeDtypeStruct((NC, nzp, D), jnp.float32)),
        mesh=_mesh,
        scratch_types=[[pltpu.VMEM((1, 128), jnp.int32)] * nbuf,
                       [pltpu.VMEM((1, 128), jnp.int32)] * nbuf,
                       pltpu.VMEM((1, 128), jnp.int32),
                       pltpu.VMEM((8, L), jnp.int32),
                       pltpu.VMEM((8, L), jnp.float32),
                       [pltpu.VMEM((128, D), jnp.float32)] * nbuf,
                       [pltpu.VMEM((128, D), jnp.float32)] * nbuf,
                       pltpu.VMEM_SHARED((ndst, D), jnp.float32),
                       pltpu.VMEM_SHARED((nzp, D), jnp.float32),
                       [pltpu.SemaphoreType.DMA] * nbuf,
                       [pltpu.SemaphoreType.DMA] * nbuf],
    )
    def kBC(h, sumh, src2d, dst2d, outp, zp,
            sis, dis, zi, lo78, ebuf, rhs, rss, out_sh, z_sh, semAs, semBs):
        c, s = lax.axis_index("c"), lax.axis_index("s")
        w = s * NC + c
        rh0 = rhs[0]
        _zero_rows(rh0, 64)
        _zero_shared(rh0, out_sh, s, ndst // NSC)
        pltpu.sync_copy(rh0.at[pl.ds(0, nzs)], z_sh.at[pl.ds(s * nzs, nzs)])
        plsc.subcore_barrier()
        lanes = lax.broadcasted_iota(jnp.int32, (L,), 0)

        def load_and_fire(b, row):
            pltpu.sync_copy(src2d.at[pl.ds(row, 1)], sis[b])
            pltpu.sync_copy(dst2d.at[pl.ds(row, 1)], dis[b])
            pltpu.async_copy(h.at[sis[b].at[0]], rhs[b], semAs[b])
            pltpu.async_copy(sumh.at[dis[b].at[0]], rss[b], semBs[b])

        def process(b):
            si, di, rh, rs = sis[b], dis[b], rhs[b], rss[b]
            pltpu.make_async_copy(h.at[si.at[0]], rh, semAs[b]).wait()
            pltpu.make_async_copy(sumh.at[di.at[0]], rs, semBs[b]).wait()
            for j in range(128 // L):
                dv = di[0, pl.ds(j * L, L)]
                zi[0, pl.ds(j * L, L)] = jnp.right_shift(dv, 7)
                lo78[j, pl.ds(0, L)] = dv & 127

            def group(g, _):
                onehots = []
                for k2 in range(L):
                    kk = g * L + k2
                    acc0 = rh[kk, pl.ds(0, L)] * rs[kk, pl.ds(0, L)]
                    acc1 = rh[kk, pl.ds(4 * L, L)] * rs[kk, pl.ds(4 * L, L)]
                    for j in range(1, 4):
                        acc0 = acc0 + (rh[kk, pl.ds(j * L, L)] *
                                       rs[kk, pl.ds(j * L, L)])
                        acc1 = acc1 + (rh[kk, pl.ds((j + 4) * L, L)] *
                                       rs[kk, pl.ds((j + 4) * L, L)])
                    acc = acc0 + acc1
                    e = [acc[l] for l in range(L)]
                    while len(e) > 1:
                        e = [e[i] + e[i + 1] for i in range(0, len(e), 2)]
                    onehots.append(jnp.where(lanes == k2, e[0], 0.0))
                while len(onehots) > 1:
                    onehots = [onehots[i] + onehots[i + 1]
                               for i in range(0, len(onehots), 2)]
                ev = jnp.exp(onehots[0])
                ebuf[g, pl.ds(0, L)] = ev
                for k2 in range(L):
                    kk = g * L + k2
                    e = ev[k2]
                    for j in range(D // L):
                        rh[kk, pl.ds(j * L, L)] = rh[kk, pl.ds(j * L, L)] * e
                return 0
            lax.fori_loop(0, 128 // L, group, 0)

            def zgroup(g, _):
                ev = ebuf[g, pl.ds(0, L)]
                lv = lo78[g, pl.ds(0, L)]
                for k2 in range(L):
                    kk = g * L + k2
                    e = ev[k2]
                    low7 = lv[k2]
                    for j in range(D // L):
                        rs[kk, pl.ds(j * L, L)] = jnp.where(
                            lanes + (j * L) == low7, e, 0.0)
                return 0
            lax.fori_loop(0, 128 // L, zgroup, 0)
            pltpu.async_copy(rh, out_sh.at[di.at[0]], semAs[b], add=True)
            pltpu.async_copy(rs, z_sh.at[zi.at[0]], semBs[b], add=True)
            pltpu.make_async_copy(rh, out_sh.at[di.at[0]], semAs[b]).wait()
            pltpu.make_async_copy(rs, z_sh.at[zi.at[0]], semBs[b]).wait()

        base = w * cpw
        if not pipelined:
            def step(t, _):
                load_and_fire(0, base + t)
                process(0)
                return 0
            lax.fori_loop(0, cpw, step, 0)
        else:
            load_and_fire(0, base)

            def pair(t2, _):
                a = base + 2 * t2
                load_and_fire(1, a + 1)
                process(0)

                @pl.when(t2 + 1 < cpw // 2)
                def _():
                    load_and_fire(0, a + 2)
                process(1)
                return 0
            lax.fori_loop(0, cpw // 2, pair, 0)
        plsc.subcore_barrier()
        _writeback(rh0, out_sh, outp, c, s, ndst // NSC)
        pltpu.sync_copy(z_sh.at[pl.ds(s * nzs, nzs)], rh0.at[pl.ds(0, nzs)])
        pltpu.sync_copy(rh0.at[pl.ds(0, nzs)], zp.at[c, pl.ds(s * nzs, nzs)])

    return kBC


_k2 = _make_passA(NTp, E1p // 128)
_k3 = _make_combine(NTp)
_k4 = _make_passBC(NTp, E1p // 128)
_c23 = _make_combine(NUp)


@functools.partial(
    pl.kernel,
    out_type=(jax.ShapeDtypeStruct((NC, NUp, D), jnp.float32),
              jax.ShapeDtypeStruct((NC, NIp, D), jnp.float32)),
    mesh=_mesh,
    scratch_types=[pltpu.VMEM((1, 128), jnp.int32),
                   pltpu.VMEM((1, 128), jnp.int32),
                   pltpu.VMEM((1, 128), jnp.int32),
                   pltpu.VMEM((1, 128), jnp.int32),
                   pltpu.VMEM((128, D), jnp.float32),
                   pltpu.VMEM((128, D), jnp.float32),
                   pltpu.VMEM_SHARED((NUp, D), jnp.float32),
                   pltpu.VMEM_SHARED((NIp, D), jnp.float32),
                   pltpu.SemaphoreType.DMA,
                   pltpu.SemaphoreType.DMA],
)
def _k6(hu, hi, src2_2d, dst2_2d, src3_2d, dst3_2d, accu, acci,
        si0, di0, si1, di1, rows0, rows1, accu_sh, acci_sh, semA, semB):
    c, s = lax.axis_index("c"), lax.axis_index("s")
    w = s * NC + c
    cpw = (E2p // 128) // NW
    _zero_rows(rows0, 64)
    _zero_shared(rows0, accu_sh, s, NUp // NSC)
    _zero_shared(rows0, acci_sh, s, NIp // NSC)
    plsc.subcore_barrier()

    def scan_edges(tbl, src2d, dst2d, sh):
        base = w * cpw
        pltpu.sync_copy(src2d.at[pl.ds(base, 1)], si0)
        pltpu.sync_copy(dst2d.at[pl.ds(base, 1)], di0)
        pltpu.async_copy(tbl.at[si0.at[0]], rows0, semA)

        def pair(t2, _):
            a = base + 2 * t2
            pltpu.sync_copy(src2d.at[pl.ds(a + 1, 1)], si1)
            pltpu.sync_copy(dst2d.at[pl.ds(a + 1, 1)], di1)
            pltpu.async_copy(tbl.at[si1.at[0]], rows1, semB)
            pltpu.make_async_copy(tbl.at[si0.at[0]], rows0, semA).wait()
            pltpu.sync_copy(rows0, sh.at[di0.at[0]], add=True)

            @pl.when(t2 + 1 < cpw // 2)
            def _():
                pltpu.sync_copy(src2d.at[pl.ds(a + 2, 1)], si0)
                pltpu.sync_copy(dst2d.at[pl.ds(a + 2, 1)], di0)
                pltpu.async_copy(tbl.at[si0.at[0]], rows0, semA)
            pltpu.make_async_copy(tbl.at[si1.at[0]], rows1, semB).wait()
            pltpu.sync_copy(rows1, sh.at[di1.at[0]], add=True)
            return 0
        lax.fori_loop(0, cpw // 2, pair, 0)

    scan_edges(hu, src2_2d, dst2_2d, accu_sh)
    scan_edges(hi, src3_2d, dst3_2d, acci_sh)
    plsc.subcore_barrier()
    _writeback(rows0, accu_sh, accu, c, s, NUp // NSC)
    _writeback(rows0, acci_sh, acci, c, s, NIp // NSC)


_k8u = _make_passBC(NUp, E2p // 128, pipelined=True)
_k8i = _make_passBC(NIp, E3p // 128, pipelined=True)


def _gelu_exact(x):
    return 0.5 * x * (1.0 + lax.erf(x * 0.7071067811865476))


def _k5_body(op_ref, zp_ref, w_ref, b_ref, wu_ref, wi_ref, hu_ref, hi_ref):
    t = op_ref[0] + op_ref[1]
    z = zp_ref[0] + zp_ref[1]
    tn = t / (z + 1e-9)
    tf = jnp.dot(tn, w_ref[...].T, preferred_element_type=jnp.float32) + b_ref[...]
    hu_ref[...] = _gelu_exact(tf * wu_ref[...])
    hi_ref[...] = _gelu_exact(tf * wi_ref[...])


def _k5(outp, zp2d, w, b2d, wu_rows, wi_rows):
    nblk = NTp // 128
    return pl.pallas_call(
        _k5_body,
        grid=(nblk,),
        in_specs=[
            pl.BlockSpec((NC, 128, D), lambda i: (0, i, 0)),
            pl.BlockSpec((NC, 128, 1), lambda i: (0, i, 0)),
            pl.BlockSpec((D, D), lambda i: (0, 0)),
            pl.BlockSpec((1, D), lambda i: (0, 0)),
            pl.BlockSpec((128, D), lambda i: (i, 0)),
            pl.BlockSpec((128, D), lambda i: (i, 0)),
        ],
        out_specs=[pl.BlockSpec((128, D), lambda i: (i, 0)),
                   pl.BlockSpec((128, D), lambda i: (i, 0))],
        out_shape=[jax.ShapeDtypeStruct((NTp, D), jnp.float32),
                   jax.ShapeDtypeStruct((NTp, D), jnp.float32)],
    )(outp, zp2d, w, b2d, wu_rows, wi_rows)


def _k9_body(up_ref, zu_ref, ip_ref, zi_ref, wu_ref, bu_ref, wi_ref, bi_ref,
             uf_ref, if_ref):
    u = (up_ref[0] + up_ref[1]) / (zu_ref[0] + zu_ref[1] + 1e-9)
    uf_ref[...] = jnp.dot(u, wu_ref[...].T,
                          preferred_element_type=jnp.float32) + bu_ref[...]
    v = (ip_ref[0] + ip_ref[1]) / (zi_ref[0] + zi_ref[1] + 1e-9)
    if_ref[...] = jnp.dot(v, wi_ref[...].T,
                          preferred_element_type=jnp.float32) + bi_ref[...]


def _k9(up, zu2d, ip, zi2d, u_w, u_b2d, i_w, i_b2d):
    nblk = NUp // 128
    return pl.pallas_call(
        _k9_body,
        grid=(nblk,),
        in_specs=[
            pl.BlockSpec((NC, 128, D), lambda i: (0, i, 0)),
            pl.BlockSpec((NC, 128, 1), lambda i: (0, i, 0)),
            pl.BlockSpec((NC, 128, D), lambda i: (0, i, 0)),
            pl.BlockSpec((NC, 128, 1), lambda i: (0, i, 0)),
            pl.BlockSpec((D, D), lambda i: (0, 0)),
            pl.BlockSpec((1, D), lambda i: (0, 0)),
            pl.BlockSpec((D, D), lambda i: (0, 0)),
            pl.BlockSpec((1, D), lambda i: (0, 0)),
        ],
        out_specs=[pl.BlockSpec((128, D), lambda i: (i, 0)),
                   pl.BlockSpec((128, D), lambda i: (i, 0))],
        out_shape=[jax.ShapeDtypeStruct((NUp, D), jnp.float32),
                   jax.ShapeDtypeStruct((NIp, D), jnp.float32)],
    )(up, zu2d, ip, zi2d, u_w, u_b2d, i_w, i_b2d)


def _pad_idx(x, n, mod):
    extra = n - x.shape[0]
    fill = jnp.arange(extra, dtype=jnp.int32) % mod
    return jnp.concatenate([x.astype(jnp.int32), fill])


def _pad_dst(x, n, real, padspace):
    extra = n - x.shape[0]
    fill = real + (jnp.arange(extra, dtype=jnp.int32) % padspace)
    return jnp.concatenate([x.astype(jnp.int32), fill])


def kernel(emb_table, sentence_w1, sent_lin_w, sent_lin_b, user_lin_w,
           user_lin_b, item_lin_w, item_lin_b, topic_user_w, topic_item_w,
           sentence_ids, stid_sent, src1, dst1, stid_user, src2, dst2,
           stid_item, src3, dst3):
    sids2d = _pad_idx(sentence_ids, NSp, VOCAB).reshape(-1, 128)
    stid2d = _pad_idx(stid_sent, NSp, 1024).reshape(-1, 128)
    su2d = _pad_idx(stid_user, NTp, 1024).reshape(-1, 128)
    si2d = _pad_idx(stid_item, NTp, 1024).reshape(-1, 128)
    src1_2d = _pad_idx(src1, E1p, NS).reshape(-1, 128)
    dst1_2d = _pad_dst(dst1, E1p, NT, NTp - NT).reshape(-1, 128)
    src2_2d = _pad_idx(src2, E2p, NT).reshape(-1, 128)
    dst2_2d = _pad_dst(dst2, E2p, NU, NUp - NU).reshape(-1, 128)
    src3_2d = _pad_idx(src3, E3p, NT).reshape(-1, 128)
    dst3_2d = _pad_dst(dst3, E3p, NI, NIp - NI).reshape(-1, 128)

    h, wu_rows, wi_rows = _k1(emb_table, sentence_w1, topic_user_w,
                              topic_item_w, sids2d, stid2d, su2d, si2d)

    sumh_p = _k2(h, src1_2d, dst1_2d)
    sumh = _k3(sumh_p)
    outp, zp = _k4(h, sumh, src1_2d, dst1_2d)
    zp2d = zp.reshape(NC, -1)[:, :NTp].reshape(NC, NTp, 1)
    hu, hi = _k5(outp, zp2d, sent_lin_w, sent_lin_b.reshape(1, D),
                 wu_rows, wi_rows)

    accu_p, acci_p = _k6(hu, hi, src2_2d, dst2_2d, src3_2d, dst3_2d)
    sumh2 = _c23(accu_p)
    sumh3 = _c23(acci_p)
    up, zu = _k8u(hu, sumh2, src2_2d, dst2_2d)
    ip, zi = _k8i(hi, sumh3, src3_2d, dst3_2d)
    zu2d = zu.reshape(NC, -1)[:, :NUp].reshape(NC, NUp, 1)
    zi2d = zi.reshape(NC, -1)[:, :NIp].reshape(NC, NIp, 1)
    user_feat, item_feat = _k9(up, zu2d, ip, zi2d,
                               user_lin_w, user_lin_b.reshape(1, D),
                               item_lin_w, item_lin_b.reshape(1, D))
    return (user_feat[:NU], item_feat[:NI])


# speedup vs baseline: 1.6462x; 1.0293x over previous
import functools
import jax
import jax.numpy as jnp
from jax import lax
from jax.experimental import pallas as pl
from jax.experimental.pallas import tpu as pltpu, tpu_sc as plsc

NS = 50000
NT = 10000
NU = 5000
NI = 5000
E1 = 500000
E2 = 320000
E3 = 320000
VOCAB = 200000
D = 128
L = 16
NC = 2
NSC = 16
NW = NC * NSC

NSp = 50176
NTp = 10240
NUp = 5120
NIp = 5120
E1p = 507904
E2p = 327680
E3p = 327680

_mesh = plsc.VectorSubcoreMesh(core_axis_name="c", subcore_axis_name="s")


def _wid():
    return lax.axis_index("s") * NC + lax.axis_index("c")


def _zero_rows(buf, nrows):
    def zr(i, _):
        for j in range(D // L):
            buf[i, pl.ds(j * L, L)] = jnp.zeros((L,), jnp.float32)
        return 0
    lax.fori_loop(0, nrows, zr, 0)


def _zero_shared(buf, sh, s, rows_per_sub):
    if rows_per_sub <= 64:
        pltpu.sync_copy(buf.at[pl.ds(0, rows_per_sub)],
                        sh.at[pl.ds(s * rows_per_sub, rows_per_sub)])
    else:
        def zc(i, _):
            pltpu.sync_copy(buf.at[pl.ds(0, 64)],
                            sh.at[pl.ds(s * rows_per_sub + i * 64, 64)])
            return 0
        lax.fori_loop(0, rows_per_sub // 64, zc, 0)


def _writeback(buf, sh, out, c, s, rows_per_sub):
    if rows_per_sub <= 64:
        pltpu.sync_copy(sh.at[pl.ds(s * rows_per_sub, rows_per_sub)],
                        buf.at[pl.ds(0, rows_per_sub)])
        pltpu.sync_copy(buf.at[pl.ds(0, rows_per_sub)],
                        out.at[c, pl.ds(s * rows_per_sub, rows_per_sub)])
    else:
        def wc(i, _):
            base = s * rows_per_sub + i * 64
            pltpu.sync_copy(sh.at[pl.ds(base, 64)], buf.at[pl.ds(0, 64)])
            pltpu.sync_copy(buf.at[pl.ds(0, 64)], out.at[c, pl.ds(base, 64)])
            return 0
        lax.fori_loop(0, rows_per_sub // 64, wc, 0)


@functools.partial(
    pl.kernel,
    out_type=(jax.ShapeDtypeStruct((NSp, D), jnp.float32),
              jax.ShapeDtypeStruct((NTp, D), jnp.float32),
              jax.ShapeDtypeStruct((NTp, D), jnp.float32)),
    mesh=_mesh,
    scratch_types=[pltpu.VMEM((1, 128), jnp.int32),
                   pltpu.VMEM((1, 128), jnp.int32),
                   pltpu.VMEM((128, D), jnp.float32),
                   pltpu.VMEM((128, D), jnp.float32),
                   pltpu.SemaphoreType.DMA,
                   pltpu.SemaphoreType.DMA],
)
def _k1(emb, w1, tuw, tiw, sids2d, stid2d, su2d, si2d,
        h_out, wu_out, wi_out, ia, ib, ra, rb, semA, semB):
    w = _wid()
    nch_h = NSp // 128

    def hchunk(t, _):
        ch = t * NW + w

        @pl.when(ch < nch_h)
        def _():
            pltpu.sync_copy(sids2d.at[pl.ds(ch, 1)], ia)
            pltpu.sync_copy(stid2d.at[pl.ds(ch, 1)], ib)
            pltpu.async_copy(emb.at[ia.at[0]], ra, semA)
            pltpu.async_copy(w1.at[ib.at[0]], rb, semB)
            pltpu.make_async_copy(emb.at[ia.at[0]], ra, semA).wait()
            pltpu.make_async_copy(w1.at[ib.at[0]], rb, semB).wait()

            def mrow(kk, _):
                for j in range(D // L):
                    ra[kk, pl.ds(j * L, L)] = (ra[kk, pl.ds(j * L, L)] *
                                               rb[kk, pl.ds(j * L, L)])
                return 0
            lax.fori_loop(0, 128, mrow, 0)
            pltpu.sync_copy(ra, h_out.at[pl.ds(ch * 128, 128)])
        return 0
    lax.fori_loop(0, nch_h // NW + 1, hchunk, 0)

    nch_t = NTp // 128

    def tchunk(t, _):
        ch = t * NW + w

        @pl.when(ch < nch_t)
        def _():
            pltpu.sync_copy(su2d.at[pl.ds(ch, 1)], ia)
            pltpu.sync_copy(si2d.at[pl.ds(ch, 1)], ib)
            pltpu.async_copy(tuw.at[ia.at[0]], ra, semA)
            pltpu.async_copy(tiw.at[ib.at[0]], rb, semB)
            pltpu.make_async_copy(tuw.at[ia.at[0]], ra, semA).wait()
            pltpu.sync_copy(ra, wu_out.at[pl.ds(ch * 128, 128)])
            pltpu.make_async_copy(tiw.at[ib.at[0]], rb, semB).wait()
            pltpu.sync_copy(rb, wi_out.at[pl.ds(ch * 128, 128)])
        return 0
    lax.fori_loop(0, nch_t // NW + 1, tchunk, 0)


def _make_passA(ndst, nchunks):
    cpw = nchunks // NW

    @functools.partial(
        pl.kernel,
        out_type=jax.ShapeDtypeStruct((NC, ndst, D), jnp.float32),
        mesh=_mesh,
        scratch_types=[pltpu.VMEM((1, 128), jnp.int32),
                       pltpu.VMEM((1, 128), jnp.int32),
                       pltpu.VMEM((1, 128), jnp.int32),
                       pltpu.VMEM((1, 128), jnp.int32),
                       pltpu.VMEM((128, D), jnp.float32),
                       pltpu.VMEM((128, D), jnp.float32),
                       pltpu.VMEM_SHARED((ndst, D), jnp.float32),
                       pltpu.SemaphoreType.DMA,
                       pltpu.SemaphoreType.DMA],
    )
    def kA(h, src2d, dst2d, accp,
           si0, di0, si1, di1, rows0, rows1, acc_sh, semA, semB):
        c, s = lax.axis_index("c"), lax.axis_index("s")
        w = s * NC + c
        _zero_rows(rows0, 64)
        _zero_shared(rows0, acc_sh, s, ndst // NSC)
        plsc.subcore_barrier()
        base = w * cpw
        pltpu.sync_copy(src2d.at[pl.ds(base, 1)], si0)
        pltpu.sync_copy(dst2d.at[pl.ds(base, 1)], di0)
        pltpu.async_copy(h.at[si0.at[0]], rows0, semA)

        def pair(t2, _):
            a = base + 2 * t2
            pltpu.sync_copy(src2d.at[pl.ds(a + 1, 1)], si1)
            pltpu.sync_copy(dst2d.at[pl.ds(a + 1, 1)], di1)
            pltpu.async_copy(h.at[si1.at[0]], rows1, semB)
            pltpu.make_async_copy(h.at[si0.at[0]], rows0, semA).wait()
            pltpu.sync_copy(rows0, acc_sh.at[di0.at[0]], add=True)

            @pl.when(t2 + 1 < cpw // 2)
            def _():
                pltpu.sync_copy(src2d.at[pl.ds(a + 2, 1)], si0)
                pltpu.sync_copy(dst2d.at[pl.ds(a + 2, 1)], di0)
                pltpu.async_copy(h.at[si0.at[0]], rows0, semA)
            pltpu.make_async_copy(h.at[si1.at[0]], rows1, semB).wait()
            pltpu.sync_copy(rows1, acc_sh.at[di1.at[0]], add=True)
            return 0
        lax.fori_loop(0, cpw // 2, pair, 0)
        plsc.subcore_barrier()
        _writeback(rows0, acc_sh, accp, c, s, ndst // NSC)

    return kA


def _make_combine(ndst):
    nblk = ndst // 512

    def body(p_ref, o_ref):
        o_ref[...] = p_ref[0] + p_ref[1]

    def kC(accp):
        return pl.pallas_call(
            body,
            grid=(nblk,),
            in_specs=[pl.BlockSpec((NC, 512, D), lambda i: (0, i, 0))],
            out_specs=pl.BlockSpec((512, D), lambda i: (i, 0)),
            out_shape=jax.ShapeDtypeStruct((ndst, D), jnp.float32),
        )(accp)

    return kC


def _make_passBC(ndst, nchunks, pipelined=False):
    cpw = nchunks // NW
    nz = ndst // 128
    nzs = 8
    nzp = nzs * NSC
    assert nz <= nzp
    nbuf = 2 if pipelined else 1

    @functools.partial(
        pl.kernel,
        out_type=(jax.ShapeDtypeStruct((NC, ndst, D), jnp.float32),
                  jax.ShapeDtypeStruct((NC, nzp, D), jnp.float32)),
        mesh=_mesh,
        scratch_types=[[pltpu.VMEM((1, 128), jnp.int32)] * 2,
                       [pltpu.VMEM((1, 128), jnp.int32)] * 2,
                       pltpu.VMEM((1, 128), jnp.int32),
                       pltpu.VMEM((8, L), jnp.int32),
                       pltpu.VMEM((8, L), jnp.float32),
                       [pltpu.VMEM((128, D), jnp.float32)] * nbuf,
                       [pltpu.VMEM((128, D), jnp.float32)] * nbuf,
                       pltpu.VMEM_SHARED((ndst, D), jnp.float32),
                       pltpu.VMEM_SHARED((nzp, D), jnp.float32),
                       [pltpu.SemaphoreType.DMA] * nbuf,
                       [pltpu.SemaphoreType.DMA] * nbuf],
    )
    def kBC(h, sumh, src2d, dst2d, outp, zp,
            sis, dis, zi, lo78, ebuf, rhs, rss, out_sh, z_sh, semAs, semBs):
        c, s = lax.axis_index("c"), lax.axis_index("s")
        w = s * NC + c
        rh0 = rhs[0]
        _zero_rows(rh0, 64)
        _zero_shared(rh0, out_sh, s, ndst // NSC)
        pltpu.sync_copy(rh0.at[pl.ds(0, nzs)], z_sh.at[pl.ds(s * nzs, nzs)])
        plsc.subcore_barrier()
        lanes = lax.broadcasted_iota(jnp.int32, (L,), 0)

        def load_idx(ib, row):
            pltpu.sync_copy(src2d.at[pl.ds(row, 1)], sis[ib])
            pltpu.sync_copy(dst2d.at[pl.ds(row, 1)], dis[ib])

        def fire(db, ib):
            pltpu.async_copy(h.at[sis[ib].at[0]], rhs[db], semAs[db])
            pltpu.async_copy(sumh.at[dis[ib].at[0]], rss[db], semBs[db])

        def process(db, ib):
            si, di, rh, rs = sis[ib], dis[ib], rhs[db], rss[db]
            pltpu.make_async_copy(h.at[si.at[0]], rh, semAs[db]).wait()
            pltpu.make_async_copy(sumh.at[di.at[0]], rs, semBs[db]).wait()
            for j in range(128 // L):
                dv = di[0, pl.ds(j * L, L)]
                zi[0, pl.ds(j * L, L)] = jnp.right_shift(dv, 7)
                lo78[j, pl.ds(0, L)] = dv & 127

            def group(g, _):
                onehots = []
                for k2 in range(L):
                    kk = g * L + k2
                    acc0 = rh[kk, pl.ds(0, L)] * rs[kk, pl.ds(0, L)]
                    acc1 = rh[kk, pl.ds(4 * L, L)] * rs[kk, pl.ds(4 * L, L)]
                    for j in range(1, 4):
                        acc0 = acc0 + (rh[kk, pl.ds(j * L, L)] *
                                       rs[kk, pl.ds(j * L, L)])
                        acc1 = acc1 + (rh[kk, pl.ds((j + 4) * L, L)] *
                                       rs[kk, pl.ds((j + 4) * L, L)])
                    acc = acc0 + acc1
                    e = [acc[l] for l in range(L)]
                    while len(e) > 1:
                        e = [e[i] + e[i + 1] for i in range(0, len(e), 2)]
                    onehots.append(jnp.where(lanes == k2, e[0], 0.0))
                while len(onehots) > 1:
                    onehots = [onehots[i] + onehots[i + 1]
                               for i in range(0, len(onehots), 2)]
                ev = jnp.exp(onehots[0])
                ebuf[g, pl.ds(0, L)] = ev
                for k2 in range(L):
                    kk = g * L + k2
                    e = ev[k2]
                    for j in range(D // L):
                        rh[kk, pl.ds(j * L, L)] = rh[kk, pl.ds(j * L, L)] * e
                return 0
            lax.fori_loop(0, 128 // L, group, 0)

            def zgroup(g, _):
                ev = ebuf[g, pl.ds(0, L)]
                lv = lo78[g, pl.ds(0, L)]
                for k2 in range(L):
                    kk = g * L + k2
                    e = ev[k2]
                    low7 = lv[k2]
                    for j in range(D // L):
                        rs[kk, pl.ds(j * L, L)] = jnp.where(
                            lanes + (j * L) == low7, e, 0.0)
                return 0
            lax.fori_loop(0, 128 // L, zgroup, 0)
            pltpu.async_copy(rh, out_sh.at[di.at[0]], semAs[db], add=True)
            pltpu.async_copy(rs, z_sh.at[zi.at[0]], semBs[db], add=True)
            pltpu.make_async_copy(rh, out_sh.at[di.at[0]], semAs[db]).wait()
            pltpu.make_async_copy(rs, z_sh.at[zi.at[0]], semBs[db]).wait()

        base = w * cpw
        if not pipelined:
            load_idx(0, base)

            def pair(t2, _):
                a = base + 2 * t2
                fire(0, 0)
                load_idx(1, a + 1)
                process(0, 0)
                fire(0, 1)

                @pl.when(t2 + 1 < cpw // 2)
                def _():
                    load_idx(0, a + 2)
                process(0, 1)
                return 0
            lax.fori_loop(0, cpw // 2, pair, 0)
        else:
            load_idx(0, base)
            fire(0, 0)

            def pair(t2, _):
                a = base + 2 * t2
                load_idx(1, a + 1)
                fire(1, 1)
                process(0, 0)

                @pl.when(t2 + 1 < cpw // 2)
                def _():
                    load_idx(0, a + 2)
                    fire(0, 0)
                process(1, 1)
                return 0
            lax.fori_loop(0, cpw // 2, pair, 0)
        plsc.subcore_barrier()
        _writeback(rh0, out_sh, outp, c, s, ndst // NSC)
        pltpu.sync_copy(z_sh.at[pl.ds(s * nzs, nzs)], rh0.at[pl.ds(0, nzs)])
        pltpu.sync_copy(rh0.at[pl.ds(0, nzs)], zp.at[c, pl.ds(s * nzs, nzs)])

    return kBC


_k2 = _make_passA(NTp, E1p // 128)
_k3 = _make_combine(NTp)
_k4 = _make_passBC(NTp, E1p // 128)
_c23 = _make_combine(NUp)


@functools.partial(
    pl.kernel,
    out_type=(jax.ShapeDtypeStruct((NC, NUp, D), jnp.float32),
              jax.ShapeDtypeStruct((NC, NIp, D), jnp.float32)),
    mesh=_mesh,
    scratch_types=[pltpu.VMEM((1, 128), jnp.int32),
                   pltpu.VMEM((1, 128), jnp.int32),
                   pltpu.VMEM((1, 128), jnp.int32),
                   pltpu.VMEM((1, 128), jnp.int32),
                   pltpu.VMEM((128, D), jnp.float32),
                   pltpu.VMEM((128, D), jnp.float32),
                   pltpu.VMEM_SHARED((NUp, D), jnp.float32),
                   pltpu.VMEM_SHARED((NIp, D), jnp.float32),
                   pltpu.SemaphoreType.DMA,
                   pltpu.SemaphoreType.DMA],
)
def _k6(hu, hi, src2_2d, dst2_2d, src3_2d, dst3_2d, accu, acci,
        si0, di0, si1, di1, rows0, rows1, accu_sh, acci_sh, semA, semB):
    c, s = lax.axis_index("c"), lax.axis_index("s")
    w = s * NC + c
    cpw = (E2p // 128) // NW
    _zero_rows(rows0, 64)
    _zero_shared(rows0, accu_sh, s, NUp // NSC)
    _zero_shared(rows0, acci_sh, s, NIp // NSC)
    plsc.subcore_barrier()

    def scan_edges(tbl, src2d, dst2d, sh):
        base = w * cpw
        pltpu.sync_copy(src2d.at[pl.ds(base, 1)], si0)
        pltpu.sync_copy(dst2d.at[pl.ds(base, 1)], di0)
        pltpu.async_copy(tbl.at[si0.at[0]], rows0, semA)

        def pair(t2, _):
            a = base + 2 * t2
            pltpu.sync_copy(src2d.at[pl.ds(a + 1, 1)], si1)
            pltpu.sync_copy(dst2d.at[pl.ds(a + 1, 1)], di1)
            pltpu.async_copy(tbl.at[si1.at[0]], rows1, semB)
            pltpu.make_async_copy(tbl.at[si0.at[0]], rows0, semA).wait()
            pltpu.sync_copy(rows0, sh.at[di0.at[0]], add=True)

            @pl.when(t2 + 1 < cpw // 2)
            def _():
                pltpu.sync_copy(src2d.at[pl.ds(a + 2, 1)], si0)
                pltpu.sync_copy(dst2d.at[pl.ds(a + 2, 1)], di0)
                pltpu.async_copy(tbl.at[si0.at[0]], rows0, semA)
            pltpu.make_async_copy(tbl.at[si1.at[0]], rows1, semB).wait()
            pltpu.sync_copy(rows1, sh.at[di1.at[0]], add=True)
            return 0
        lax.fori_loop(0, cpw // 2, pair, 0)

    scan_edges(hu, src2_2d, dst2_2d, accu_sh)
    scan_edges(hi, src3_2d, dst3_2d, acci_sh)
    plsc.subcore_barrier()
    _writeback(rows0, accu_sh, accu, c, s, NUp // NSC)
    _writeback(rows0, acci_sh, acci, c, s, NIp // NSC)


_k8u = _make_passBC(NUp, E2p // 128, pipelined=True)
_k8i = _make_passBC(NIp, E3p // 128, pipelined=True)


def _gelu_exact(x):
    return 0.5 * x * (1.0 + lax.erf(x * 0.7071067811865476))


def _k5_body(op_ref, zp_ref, w_ref, b_ref, wu_ref, wi_ref, hu_ref, hi_ref):
    t = op_ref[0] + op_ref[1]
    z = zp_ref[0] + zp_ref[1]
    tn = t / (z + 1e-9)
    tf = jnp.dot(tn, w_ref[...].T, preferred_element_type=jnp.float32) + b_ref[...]
    hu_ref[...] = _gelu_exact(tf * wu_ref[...])
    hi_ref[...] = _gelu_exact(tf * wi_ref[...])


def _k5(outp, zp2d, w, b2d, wu_rows, wi_rows):
    nblk = NTp // 128
    return pl.pallas_call(
        _k5_body,
        grid=(nblk,),
        in_specs=[
            pl.BlockSpec((NC, 128, D), lambda i: (0, i, 0)),
            pl.BlockSpec((NC, 128, 1), lambda i: (0, i, 0)),
            pl.BlockSpec((D, D), lambda i: (0, 0)),
            pl.BlockSpec((1, D), lambda i: (0, 0)),
            pl.BlockSpec((128, D), lambda i: (i, 0)),
            pl.BlockSpec((128, D), lambda i: (i, 0)),
        ],
        out_specs=[pl.BlockSpec((128, D), lambda i: (i, 0)),
                   pl.BlockSpec((128, D), lambda i: (i, 0))],
        out_shape=[jax.ShapeDtypeStruct((NTp, D), jnp.float32),
                   jax.ShapeDtypeStruct((NTp, D), jnp.float32)],
    )(outp, zp2d, w, b2d, wu_rows, wi_rows)


def _k9_body(up_ref, zu_ref, ip_ref, zi_ref, wu_ref, bu_ref, wi_ref, bi_ref,
             uf_ref, if_ref):
    u = (up_ref[0] + up_ref[1]) / (zu_ref[0] + zu_ref[1] + 1e-9)
    uf_ref[...] = jnp.dot(u, wu_ref[...].T,
                          preferred_element_type=jnp.float32) + bu_ref[...]
    v = (ip_ref[0] + ip_ref[1]) / (zi_ref[0] + zi_ref[1] + 1e-9)
    if_ref[...] = jnp.dot(v, wi_ref[...].T,
                          preferred_element_type=jnp.float32) + bi_ref[...]


def _k9(up, zu2d, ip, zi2d, u_w, u_b2d, i_w, i_b2d):
    nblk = NUp // 128
    return pl.pallas_call(
        _k9_body,
        grid=(nblk,),
        in_specs=[
            pl.BlockSpec((NC, 128, D), lambda i: (0, i, 0)),
            pl.BlockSpec((NC, 128, 1), lambda i: (0, i, 0)),
            pl.BlockSpec((NC, 128, D), lambda i: (0, i, 0)),
            pl.BlockSpec((NC, 128, 1), lambda i: (0, i, 0)),
            pl.BlockSpec((D, D), lambda i: (0, 0)),
            pl.BlockSpec((1, D), lambda i: (0, 0)),
            pl.BlockSpec((D, D), lambda i: (0, 0)),
            pl.BlockSpec((1, D), lambda i: (0, 0)),
        ],
        out_specs=[pl.BlockSpec((128, D), lambda i: (i, 0)),
                   pl.BlockSpec((128, D), lambda i: (i, 0))],
        out_shape=[jax.ShapeDtypeStruct((NUp, D), jnp.float32),
                   jax.ShapeDtypeStruct((NIp, D), jnp.float32)],
    )(up, zu2d, ip, zi2d, u_w, u_b2d, i_w, i_b2d)


def _pad_idx(x, n, mod):
    extra = n - x.shape[0]
    fill = jnp.arange(extra, dtype=jnp.int32) % mod
    return jnp.concatenate([x.astype(jnp.int32), fill])


def _pad_dst(x, n, real, padspace):
    extra = n - x.shape[0]
    fill = real + (jnp.arange(extra, dtype=jnp.int32) % padspace)
    return jnp.concatenate([x.astype(jnp.int32), fill])


def kernel(emb_table, sentence_w1, sent_lin_w, sent_lin_b, user_lin_w,
           user_lin_b, item_lin_w, item_lin_b, topic_user_w, topic_item_w,
           sentence_ids, stid_sent, src1, dst1, stid_user, src2, dst2,
           stid_item, src3, dst3):
    sids2d = _pad_idx(sentence_ids, NSp, VOCAB).reshape(-1, 128)
    stid2d = _pad_idx(stid_sent, NSp, 1024).reshape(-1, 128)
    su2d = _pad_idx(stid_user, NTp, 1024).reshape(-1, 128)
    si2d = _pad_idx(stid_item, NTp, 1024).reshape(-1, 128)
    src1_2d = _pad_idx(src1, E1p, NS).reshape(-1, 128)
    dst1_2d = _pad_dst(dst1, E1p, NT, NTp - NT).reshape(-1, 128)
    src2_2d = _pad_idx(src2, E2p, NT).reshape(-1, 128)
    dst2_2d = _pad_dst(dst2, E2p, NU, NUp - NU).reshape(-1, 128)
    src3_2d = _pad_idx(src3, E3p, NT).reshape(-1, 128)
    dst3_2d = _pad_dst(dst3, E3p, NI, NIp - NI).reshape(-1, 128)

    h, wu_rows, wi_rows = _k1(emb_table, sentence_w1, topic_user_w,
                              topic_item_w, sids2d, stid2d, su2d, si2d)

    sumh_p = _k2(h, src1_2d, dst1_2d)
    sumh = _k3(sumh_p)
    outp, zp = _k4(h, sumh, src1_2d, dst1_2d)
    zp2d = zp.reshape(NC, -1)[:, :NTp].reshape(NC, NTp, 1)
    hu, hi = _k5(outp, zp2d, sent_lin_w, sent_lin_b.reshape(1, D),
                 wu_rows, wi_rows)

    accu_p, acci_p = _k6(hu, hi, src2_2d, dst2_2d, src3_2d, dst3_2d)
    sumh2 = _c23(accu_p)
    sumh3 = _c23(acci_p)
    up, zu = _k8u(hu, sumh2, src2_2d, dst2_2d)
    ip, zi = _k8i(hi, sumh3, src3_2d, dst3_2d)
    zu2d = zu.reshape(NC, -1)[:, :NUp].reshape(NC, NUp, 1)
    zi2d = zi.reshape(NC, -1)[:, :NIp].reshape(NC, NIp, 1)
    user_feat, item_feat = _k9(up, zu2d, ip, zi2d,
                               user_lin_w, user_lin_b.reshape(1, D),
                               item_lin_w, item_lin_b.reshape(1, D))
    return (user_feat[:NU], item_feat[:NI])
